# Initial kernel scaffold; baseline (speedup 1.0000x reference)
import jax
import jax.numpy as jnp
from jax.experimental import pallas as pl


def kernel(atom_features, edge_features, edge_sh, edge_index, W_q, Wk1, bk1, Wk2, bk2, Wv1, bv1, Wv2, bv2, bn_gamma, bn_beta, W_e, We1, be1, We2, be2, We3, be3, ln_gamma, ln_beta):
    raise NotImplementedError("write your pallas kernel here")


# speedup vs baseline: 1.0008x; 1.0008x over previous
import jax
import jax.numpy as jnp
from jax.experimental import pallas as pl
from jax.experimental.pallas import tpu as pltpu

N = 10000
E = 320000
D = 128
H = 16


def _bn_body(x_ref, gamma_ref, beta_ref, out_ref):
    x = x_ref[...]
    mu = jnp.mean(x, axis=0, keepdims=True)
    var = jnp.mean((x - mu) * (x - mu), axis=0, keepdims=True)
    out_ref[...] = (x - mu) * jax.lax.rsqrt(var + 1e-5) * gamma_ref[...] + beta_ref[...]


def _batchnorm(x, gamma, beta):
    return pl.pallas_call(
        _bn_body,
        out_shape=jax.ShapeDtypeStruct((N, D), jnp.float32),
    )(x, gamma.reshape(1, D), beta.reshape(1, D))


def kernel(atom_features, edge_features, edge_sh, edge_index, W_q, Wk1, bk1, Wk2, bk2, Wv1, bv1, Wv2, bv2, bn_gamma, bn_beta, W_e, We1, be1, We2, be2, We3, be3, ln_gamma, ln_beta):
    dst = edge_index[0]
    src = edge_index[1]
    q = (atom_features @ W_q)[src]
    kw = jax.nn.relu(edge_features @ Wk1 + bk1) @ Wk2 + bk2
    vw = jax.nn.relu(edge_features @ Wv1 + bv1) @ Wv2 + bv2
    x_dst = atom_features[dst]
    k = x_dst * edge_sh * kw
    v = x_dst * edge_sh * vw
    attn = jnp.sum(q * k, axis=-1)
    m = jax.ops.segment_max(attn, src, num_segments=N)
    m = jnp.where(jnp.isfinite(m), m, 0.0)
    ex = jnp.exp(attn - m[src])
    s = jax.ops.segment_sum(ex, src, num_segments=N)
    alpha = ex / (s[src] + 1e-16)
    upd = jax.ops.segment_sum(v * alpha[:, None], src, num_segments=N)
    x = atom_features + upd
    x = _batchnorm(x, bn_gamma, bn_beta)
    sa = x @ W_e
    e_in = jnp.concatenate([sa[dst], sa[src], edge_features], axis=-1)
    h = jax.nn.relu(e_in @ We1 + be1)
    h = jax.nn.relu(h @ We2 + be2)
    h = h @ We3 + be3
    e = edge_features + h
    mu_e = jnp.mean(e, axis=-1, keepdims=True)
    var_e = jnp.var(e, axis=-1, keepdims=True)
    e = (e - mu_e) / jnp.sqrt(var_e + 1e-5) * ln_gamma + ln_beta
    return (x, e)


# speedup vs baseline: 3.1558x; 3.1532x over previous
import jax
import jax.numpy as jnp
from jax import lax
from jax.experimental import pallas as pl
from jax.experimental.pallas import tpu as pltpu
from jax.experimental.pallas import tpu_sc as plsc

N = 10000
E = 320000
D = 128
H = 16

NC = 2
NS = 16
NW = NC * NS
EW = E // NW
CH = 80
NCH = EW // CH
NP = 10240
L = 16

def _get_mesh():
    return plsc.VectorSubcoreMesh(core_axis_name="c", subcore_axis_name="s")


def _f32(*shape):
    return jax.ShapeDtypeStruct(shape, jnp.float32)


def _sc_gather_pair(ta, tb, ia3, ib3, dw, tc_tiling=True):
    def body(ta_h, tb_h, ia_h, ib_h, outa_h, outb_h,
             idxa, idxb, ba0, ba1, bb0, bb1,
             sga0, sga1, sgb0, sgb1, swa0, swa1, swb0, swb1):
        cid = lax.axis_index("c")
        sid = lax.axis_index("s")
        wid = cid * NS + sid
        base = wid * EW
        pltpu.sync_copy(ia_h.at[wid], idxa)
        pltpu.sync_copy(ib_h.at[wid], idxb)
        ba = (ba0, ba1)
        bb = (bb0, bb1)
        sga = (sga0, sga1)
        sgb = (sgb0, sgb1)
        swa = (swa0, swa1)
        swb = (swb0, swb1)

        def loop_body(c, carry):
            da = pltpu.async_copy(ta_h.at[idxa.at[c]], ba[0], sga[0])
            db = pltpu.async_copy(tb_h.at[idxb.at[c]], bb[0], sgb[0])
            da.wait()
            db.wait()
            wa = pltpu.async_copy(ba[0], outa_h.at[pl.ds(base + c * CH, CH)], swa[0])
            wb = pltpu.async_copy(bb[0], outb_h.at[pl.ds(base + c * CH, CH)], swb[0])
            wa.wait()
            wb.wait()
            return carry

        lax.fori_loop(0, NCH, loop_body, 0)

    return pl.kernel(
        body,
        out_type=(_f32(E, dw), _f32(E, dw)),
        mesh=_get_mesh(),
        compiler_params=pltpu.CompilerParams(
            needs_layout_passes=False, use_tc_tiling_on_sc=tc_tiling),
        scratch_types=[
            pltpu.VMEM((NCH, CH), jnp.int32),
            pltpu.VMEM((NCH, CH), jnp.int32),
            pltpu.VMEM((CH, dw), jnp.float32),
            pltpu.VMEM((CH, dw), jnp.float32),
            pltpu.VMEM((CH, dw), jnp.float32),
            pltpu.VMEM((CH, dw), jnp.float32),
        ] + [pltpu.SemaphoreType.DMA] * 8,
    )(ta, tb, ia3, ib3)


def _sc_segmax_partials(attn, src2):
    def body(attn_h, src_h, out_h, mbuf, abuf, sbuf):
        cid = lax.axis_index("c")
        sid = lax.axis_index("s")
        wid = cid * NS + sid
        pltpu.sync_copy(attn_h.at[pl.ds(wid * EW, EW)], abuf)
        pltpu.sync_copy(src_h.at[wid], sbuf)

        def zero(j, c):
            mbuf[pl.ds(j * L, L)] = jnp.full((L,), -1e30, jnp.float32)
            return c

        lax.fori_loop(0, N // L, zero, 0)
        iot = lax.broadcasted_iota(jnp.int32, (L,), 0)

        def step(j, c):
            idx = sbuf[pl.ds(j * L, L)]
            a = abuf[pl.ds(j * L, L)]
            for l in range(L):
                mk = iot == l
                cur = plsc.load_gather(mbuf, [idx], mask=mk)
                plsc.store_scatter(mbuf, [idx], jnp.maximum(cur, a), mask=mk)
            return c

        lax.fori_loop(0, EW // L, step, 0)
        pltpu.sync_copy(mbuf, out_h.at[wid])

    return pl.kernel(
        body,
        out_type=_f32(NW, N),
        mesh=_get_mesh(),
        compiler_params=pltpu.CompilerParams(needs_layout_passes=False),
        scratch_types=[
            pltpu.VMEM((N,), jnp.float32),
            pltpu.VMEM((EW,), jnp.float32),
            pltpu.VMEM((EW,), jnp.int32),
        ],
    )(attn, src2)


def _sc_exp_segsum(attn, src2, m):
    def body(attn_h, src_h, m_h, ex_h, out_h, mfull, abuf, sbuf, ebuf, ssum):
        cid = lax.axis_index("c")
        sid = lax.axis_index("s")
        wid = cid * NS + sid
        pltpu.sync_copy(m_h, mfull)
        pltpu.sync_copy(attn_h.at[pl.ds(wid * EW, EW)], abuf)
        pltpu.sync_copy(src_h.at[wid], sbuf)

        def zero(j, c):
            ssum[pl.ds(j * L, L)] = jnp.zeros((L,), jnp.float32)
            return c

        lax.fori_loop(0, N // L, zero, 0)
        iot = lax.broadcasted_iota(jnp.int32, (L,), 0)

        def step(j, c):
            idx = sbuf[pl.ds(j * L, L)]
            a = abuf[pl.ds(j * L, L)]
            mg = plsc.load_gather(mfull, [idx])
            ex = jnp.exp(a - mg)
            ebuf[pl.ds(j * L, L)] = ex
            for l in range(L):
                mk = iot == l
                cur = plsc.load_gather(ssum, [idx], mask=mk)
                plsc.store_scatter(ssum, [idx], cur + ex, mask=mk)
            return c

        lax.fori_loop(0, EW // L, step, 0)
        pltpu.sync_copy(ebuf, ex_h.at[pl.ds(wid * EW, EW)])
        pltpu.sync_copy(ssum, out_h.at[wid])

    return pl.kernel(
        body,
        out_type=(_f32(E), _f32(NW, N)),
        mesh=_get_mesh(),
        compiler_params=pltpu.CompilerParams(needs_layout_passes=False),
        scratch_types=[
            pltpu.VMEM((N,), jnp.float32),
            pltpu.VMEM((EW,), jnp.float32),
            pltpu.VMEM((EW,), jnp.int32),
            pltpu.VMEM((EW,), jnp.float32),
            pltpu.VMEM((N,), jnp.float32),
        ],
    )(attn, src2, m)


def _sc_alpha(ex, src2, ssum, sh):
    def body(ex_h, src_h, s_h, sh_h, w_h, sfull, ebuf, sbuf, hbuf, wbuf):
        cid = lax.axis_index("c")
        sid = lax.axis_index("s")
        wid = cid * NS + sid
        pltpu.sync_copy(s_h, sfull)
        pltpu.sync_copy(ex_h.at[pl.ds(wid * EW, EW)], ebuf)
        pltpu.sync_copy(src_h.at[wid], sbuf)
        pltpu.sync_copy(sh_h.at[pl.ds(wid * EW, EW)], hbuf)

        def step(j, c):
            idx = sbuf[pl.ds(j * L, L)]
            e = ebuf[pl.ds(j * L, L)]
            s = plsc.load_gather(sfull, [idx])
            wbuf[pl.ds(j * L, L)] = hbuf[pl.ds(j * L, L)] * e / (s + 1e-16)
            return c

        lax.fori_loop(0, EW // L, step, 0)
        pltpu.sync_copy(wbuf, w_h.at[pl.ds(wid * EW, EW)])

    return pl.kernel(
        body,
        out_type=_f32(E),
        mesh=_get_mesh(),
        compiler_params=pltpu.CompilerParams(needs_layout_passes=False),
        scratch_types=[
            pltpu.VMEM((N,), jnp.float32),
            pltpu.VMEM((EW,), jnp.float32),
            pltpu.VMEM((EW,), jnp.int32),
            pltpu.VMEM((EW,), jnp.float32),
            pltpu.VMEM((EW,), jnp.float32),
        ],
    )(ex, src2, ssum, sh)


def _sc_scatter_add(yl, yr, src3):
    rows_per_tile = NP // NS
    zch = 128
    DH = D // 2

    def body(yl_h, yr_h, src_h, outl_h, outr_h, idxf, yb0, yb1, zb, acc, sl0, sl1):
        cid = lax.axis_index("c")
        sid = lax.axis_index("s")
        wid = cid * NS + sid
        base = wid * EW
        pltpu.sync_copy(src_h.at[wid], idxf)

        def zz(i, c):
            for k2 in range(DH // L):
                zb[i, pl.ds(k2 * L, L)] = jnp.zeros((L,), jnp.float32)
            return c

        lax.fori_loop(0, zch, zz, 0)
        yb = (yb0, yb1)
        sl = (sl0, sl1)

        for half in range(2):
            y_h = (yl_h, yr_h)[half]
            out_h = (outl_h, outr_h)[half]
            for k in range(rows_per_tile // zch):
                pltpu.sync_copy(
                    zb, acc.at[pl.ds(sid * rows_per_tile + k * zch, zch)])
            plsc.subcore_barrier()

            def lb(c, carry):
                pltpu.async_copy(
                    y_h.at[pl.ds(base + c * CH, CH)], yb[0], sl[0]).wait()
                pltpu.sync_copy(yb[0], acc.at[idxf.at[c]], add=True)
                return carry

            lax.fori_loop(0, NCH, lb, 0)
            plsc.subcore_barrier()
            for k in range(rows_per_tile // zch):
                r0 = sid * rows_per_tile + k * zch
                pltpu.sync_copy(acc.at[pl.ds(r0, zch)], zb)
                pltpu.sync_copy(zb, out_h.at[pl.ds(cid * NP + r0, zch)])
            if half == 0:
                lax.fori_loop(0, zch, zz, 0)

    return pl.kernel(
        body,
        out_type=(_f32(NC * NP, DH), _f32(NC * NP, DH)),
        mesh=_get_mesh(),
        compiler_params=pltpu.CompilerParams(
            needs_layout_passes=False, use_tc_tiling_on_sc=False),
        scratch_types=[
            pltpu.VMEM((NCH, CH), jnp.int32),
            pltpu.VMEM((CH, DH), jnp.float32),
            pltpu.VMEM((CH, DH), jnp.float32),
            pltpu.VMEM((zch, DH), jnp.float32),
            pltpu.VMEM_SHARED((NP, DH), jnp.float32),
            pltpu.SemaphoreType.DMA,
            pltpu.SemaphoreType.DMA,
        ],
    )(yl, yr, src3)


def _tc_q(atom, wq):
    def body(a_ref, w_ref, o_ref):
        o_ref[...] = jnp.dot(a_ref[...], w_ref[...],
                             preferred_element_type=jnp.float32)

    return pl.pallas_call(body, out_shape=_f32(N, D))(atom, wq)


def _tc_attn(q_src, x_dst, ef, sh, wk1, bk1, wk2, bk2):
    be = 4000
    grid = E // be

    def body(q_ref, x_ref, e_ref, s_ref, w1_ref, b1_ref, w2_ref, b2_ref, o_ref):
        r = jax.nn.relu(jnp.dot(e_ref[...], w1_ref[...],
                                preferred_element_type=jnp.float32) + b1_ref[...])
        g = q_ref[...] * x_ref[...]
        t = lax.dot_general(g, w2_ref[...], (((1,), (1,)), ((), ())),
                            preferred_element_type=jnp.float32)
        att = jnp.sum(r * t, axis=-1, keepdims=True)
        att = att + lax.dot_general(g, b2_ref[...], (((1,), (1,)), ((), ())),
                                    preferred_element_type=jnp.float32)
        o_ref[...] = att * s_ref[...]

    return pl.pallas_call(
        body,
        grid=(grid,),
        in_specs=[
            pl.BlockSpec((be, D), lambda i: (i, 0)),
            pl.BlockSpec((be, D), lambda i: (i, 0)),
            pl.BlockSpec((be, H), lambda i: (i, 0)),
            pl.BlockSpec((be, 1), lambda i: (i, 0)),
            pl.BlockSpec((H, H), lambda i: (0, 0)),
            pl.BlockSpec((1, H), lambda i: (0, 0)),
            pl.BlockSpec((H, D), lambda i: (0, 0)),
            pl.BlockSpec((1, D), lambda i: (0, 0)),
        ],
        out_specs=pl.BlockSpec((be, 1), lambda i: (i, 0)),
        out_shape=_f32(E, 1),
    )(q_src, x_dst, ef, sh, wk1, bk1.reshape(1, H), wk2, bk2.reshape(1, D))


def _tc_merge(parts, op):
    def body(p_ref, o_ref):
        if op == "max":
            o_ref[...] = jnp.max(p_ref[...], axis=0, keepdims=True)
        else:
            o_ref[...] = jnp.sum(p_ref[...], axis=0, keepdims=True)

    return pl.pallas_call(body, out_shape=_f32(1, N))(parts)


def _tc_values(w, x_dst, ef, wv1, bv1, wv2, bv2):
    be = 4000
    grid = E // be

    def body(w_ref, x_ref, e_ref, w1_ref, b1_ref, w2_ref, b2_ref, ol_ref, or_ref):
        s2 = jax.nn.relu(jnp.dot(e_ref[...], w1_ref[...],
                                 preferred_element_type=jnp.float32) + b1_ref[...])
        vw = jnp.dot(s2, w2_ref[...],
                     preferred_element_type=jnp.float32) + b2_ref[...]
        y = x_ref[...] * vw * w_ref[...]
        ol_ref[...] = y[:, 0:D // 2]
        or_ref[...] = y[:, D // 2:D]

    return pl.pallas_call(
        body,
        grid=(grid,),
        in_specs=[
            pl.BlockSpec((be, 1), lambda i: (i, 0)),
            pl.BlockSpec((be, D), lambda i: (i, 0)),
            pl.BlockSpec((be, H), lambda i: (i, 0)),
            pl.BlockSpec((H, H), lambda i: (0, 0)),
            pl.BlockSpec((1, H), lambda i: (0, 0)),
            pl.BlockSpec((H, D), lambda i: (0, 0)),
            pl.BlockSpec((1, D), lambda i: (0, 0)),
        ],
        out_specs=(pl.BlockSpec((be, D // 2), lambda i: (i, 0)),
                   pl.BlockSpec((be, D // 2), lambda i: (i, 0))),
        out_shape=(_f32(E, D // 2), _f32(E, D // 2)),
    )(w.reshape(E, 1), x_dst, ef, wv1, bv1.reshape(1, H), wv2, bv2.reshape(1, D))


def _tc_node_update(atom, updl, updr, gamma, beta, we):
    def body(a_ref, ul_ref, ur_ref, g_ref, b_ref, w_ref, x_ref, sa_ref):
        ul = ul_ref[...]
        ur = ur_ref[...]
        u = jnp.concatenate([ul[0:N] + ul[NP:NP + N],
                             ur[0:N] + ur[NP:NP + N]], axis=1)
        x = a_ref[...] + u
        mu = jnp.mean(x, axis=0, keepdims=True)
        d = x - mu
        var = jnp.mean(d * d, axis=0, keepdims=True)
        xn = d * lax.rsqrt(var + 1e-5) * g_ref[...] + b_ref[...]
        x_ref[...] = xn
        sa_ref[...] = jnp.dot(xn, w_ref[...], preferred_element_type=jnp.float32)

    return pl.pallas_call(
        body,
        out_shape=(_f32(N, D), _f32(N, H)),
    )(atom, updl, updr, gamma.reshape(1, D), beta.reshape(1, D), we)


def _tc_edge_mlp(ef, sa_dst, sa_src, we1, be1, we2, be2, we3, be3, lg, lb):
    be = 8000
    grid = E // be
    a1 = we1[0:H]
    a2 = we1[H:2 * H]
    a3 = we1[2 * H:3 * H]

    def body(e_ref, d_ref, s_ref, a1_ref, a2_ref, a3_ref, b1_ref,
             w2_ref, b2_ref, w3_ref, b3_ref, g_ref, bb_ref, o_ref):
        efv = e_ref[...]
        h = (jnp.dot(d_ref[...], a1_ref[...], preferred_element_type=jnp.float32)
             + jnp.dot(s_ref[...], a2_ref[...], preferred_element_type=jnp.float32)
             + jnp.dot(efv, a3_ref[...], preferred_element_type=jnp.float32)
             + b1_ref[...])
        h = jax.nn.relu(h)
        h = jax.nn.relu(jnp.dot(h, w2_ref[...],
                                preferred_element_type=jnp.float32) + b2_ref[...])
        h = jnp.dot(h, w3_ref[...], preferred_element_type=jnp.float32) + b3_ref[...]
        e = efv + h
        mu = jnp.mean(e, axis=-1, keepdims=True)
        dv = e - mu
        var = jnp.mean(dv * dv, axis=-1, keepdims=True)
        o_ref[...] = dv * lax.rsqrt(var + 1e-5) * g_ref[...] + bb_ref[...]

    wspec = pl.BlockSpec((H, H), lambda i: (0, 0))
    bspec = pl.BlockSpec((1, H), lambda i: (0, 0))
    espec = pl.BlockSpec((be, H), lambda i: (i, 0))
    return pl.pallas_call(
        body,
        grid=(grid,),
        in_specs=[espec, espec, espec, wspec, wspec, wspec, bspec,
                  wspec, bspec, wspec, bspec, bspec, bspec],
        out_specs=espec,
        out_shape=_f32(E, H),
    )(ef, sa_dst, sa_src, a1, a2, a3, be1.reshape(1, H),
      we2, be2.reshape(1, H), we3, be3.reshape(1, H),
      lg.reshape(1, H), lb.reshape(1, H))


def kernel(atom_features, edge_features, edge_sh, edge_index, W_q, Wk1, bk1,
           Wk2, bk2, Wv1, bv1, Wv2, bv2, bn_gamma, bn_beta, W_e, We1, be1,
           We2, be2, We3, be3, ln_gamma, ln_beta):
    dst = edge_index[0]
    src = edge_index[1]
    src2 = src.reshape(NW, EW)
    src3 = src.reshape(NW, NCH, CH)
    dst3 = dst.reshape(NW, NCH, CH)
    sh_flat = edge_sh.reshape(E)

    qn = _tc_q(atom_features, W_q)
    q_src, x_dst = _sc_gather_pair(qn, atom_features, src3, dst3, D)
    attn = _tc_attn(q_src, x_dst, edge_features, edge_sh, Wk1, bk1, Wk2, bk2)
    attn_flat = attn.reshape(E)
    mpart = _sc_segmax_partials(attn_flat, src2)
    m = _tc_merge(mpart, "max").reshape(N)
    ex, spart = _sc_exp_segsum(attn_flat, src2, m)
    ssum = _tc_merge(spart, "sum").reshape(N)
    w = _sc_alpha(ex, src2, ssum, sh_flat)
    yl, yr = _tc_values(w, x_dst, edge_features, Wv1, bv1, Wv2, bv2)
    updl, updr = _sc_scatter_add(yl, yr, src3)
    x_out, sa = _tc_node_update(atom_features, updl, updr, bn_gamma, bn_beta, W_e)
    sa_dst, sa_src = _sc_gather_pair(sa, sa, dst3, src3, H, tc_tiling=False)
    e_out = _tc_edge_mlp(edge_features, sa_dst, sa_src, We1, be1, We2, be2,
                         We3, be3, ln_gamma, ln_beta)
    return (x_out, e_out)


# speedup vs baseline: 3.2510x; 1.0302x over previous
import jax
import jax.numpy as jnp
from jax import lax
from jax.experimental import pallas as pl
from jax.experimental.pallas import tpu as pltpu
from jax.experimental.pallas import tpu_sc as plsc

N = 10000
E = 320000
D = 128
H = 16

NC = 2
NS = 16
NW = NC * NS
EW = E // NW
CH = 40
NCH = EW // CH
NP = 10240
L = 16

def _get_mesh():
    return plsc.VectorSubcoreMesh(core_axis_name="c", subcore_axis_name="s")


def _f32(*shape):
    return jax.ShapeDtypeStruct(shape, jnp.float32)


def _sc_gather_pair(ta, tb, ia3, ib3, dw, tc_tiling=True):
    KB = 5

    def body(ta_h, tb_h, ia_h, ib_h, outa_h, outb_h, *scr):
        idxa, idxb = scr[0], scr[1]
        ba = scr[2:2 + KB]
        bb = scr[2 + KB:2 + 2 * KB]
        sems = scr[2 + 2 * KB:]
        sga = sems[0:KB]
        sgb = sems[KB:2 * KB]
        swa = sems[2 * KB:3 * KB]
        swb = sems[3 * KB:4 * KB]
        cid = lax.axis_index("c")
        sid = lax.axis_index("s")
        wid = cid * NS + sid
        base = wid * EW
        pltpu.sync_copy(ia_h.at[wid], idxa)
        pltpu.sync_copy(ib_h.at[wid], idxb)

        def loop_body(g, carry):
            c0 = g * KB
            gd = []
            for j in range(KB):
                gd.append((
                    pltpu.async_copy(ta_h.at[idxa.at[c0 + j]], ba[j], sga[j]),
                    pltpu.async_copy(tb_h.at[idxb.at[c0 + j]], bb[j], sgb[j]),
                ))
            wd = []
            for j in range(KB):
                gd[j][0].wait()
                gd[j][1].wait()
                wd.append((
                    pltpu.async_copy(
                        ba[j], outa_h.at[pl.ds(base + (c0 + j) * CH, CH)], swa[j]),
                    pltpu.async_copy(
                        bb[j], outb_h.at[pl.ds(base + (c0 + j) * CH, CH)], swb[j]),
                ))
            for j in range(KB):
                wd[j][0].wait()
                wd[j][1].wait()
            return carry

        lax.fori_loop(0, NCH // KB, loop_body, 0)

    return pl.kernel(
        body,
        out_type=(_f32(E, dw), _f32(E, dw)),
        mesh=_get_mesh(),
        compiler_params=pltpu.CompilerParams(
            needs_layout_passes=False, use_tc_tiling_on_sc=tc_tiling),
        scratch_types=[
            pltpu.VMEM((NCH, CH), jnp.int32),
            pltpu.VMEM((NCH, CH), jnp.int32),
        ] + [pltpu.VMEM((CH, dw), jnp.float32)] * (2 * KB)
          + [pltpu.SemaphoreType.DMA] * (4 * KB),
    )(ta, tb, ia3, ib3)


def _sc_segmax_partials(attn, src2):
    def body(attn_h, src_h, out_h, mbuf, abuf, sbuf):
        cid = lax.axis_index("c")
        sid = lax.axis_index("s")
        wid = cid * NS + sid
        pltpu.sync_copy(attn_h.at[pl.ds(wid * EW, EW)], abuf)
        pltpu.sync_copy(src_h.at[wid], sbuf)

        def zero(j, c):
            mbuf[pl.ds(j * L, L)] = jnp.full((L,), -1e30, jnp.float32)
            return c

        lax.fori_loop(0, N // L, zero, 0)
        iot = lax.broadcasted_iota(jnp.int32, (L,), 0)

        def step(j, c):
            idx = sbuf[pl.ds(j * L, L)]
            a = abuf[pl.ds(j * L, L)]
            for l in range(L):
                mk = iot == l
                cur = plsc.load_gather(mbuf, [idx], mask=mk)
                plsc.store_scatter(mbuf, [idx], jnp.maximum(cur, a), mask=mk)
            return c

        lax.fori_loop(0, EW // L, step, 0)
        pltpu.sync_copy(mbuf, out_h.at[wid])

    return pl.kernel(
        body,
        out_type=_f32(NW, N),
        mesh=_get_mesh(),
        compiler_params=pltpu.CompilerParams(needs_layout_passes=False),
        scratch_types=[
            pltpu.VMEM((N,), jnp.float32),
            pltpu.VMEM((EW,), jnp.float32),
            pltpu.VMEM((EW,), jnp.int32),
        ],
    )(attn, src2)


def _sc_exp_segsum(attn, src2, m):
    def body(attn_h, src_h, m_h, ex_h, out_h, mfull, abuf, sbuf, ebuf, ssum):
        cid = lax.axis_index("c")
        sid = lax.axis_index("s")
        wid = cid * NS + sid
        pltpu.sync_copy(m_h, mfull)
        pltpu.sync_copy(attn_h.at[pl.ds(wid * EW, EW)], abuf)
        pltpu.sync_copy(src_h.at[wid], sbuf)

        def zero(j, c):
            ssum[pl.ds(j * L, L)] = jnp.zeros((L,), jnp.float32)
            return c

        lax.fori_loop(0, N // L, zero, 0)
        iot = lax.broadcasted_iota(jnp.int32, (L,), 0)

        def step(j, c):
            idx = sbuf[pl.ds(j * L, L)]
            a = abuf[pl.ds(j * L, L)]
            mg = plsc.load_gather(mfull, [idx])
            ex = jnp.exp(a - mg)
            ebuf[pl.ds(j * L, L)] = ex
            for l in range(L):
                mk = iot == l
                cur = plsc.load_gather(ssum, [idx], mask=mk)
                plsc.store_scatter(ssum, [idx], cur + ex, mask=mk)
            return c

        lax.fori_loop(0, EW // L, step, 0)
        pltpu.sync_copy(ebuf, ex_h.at[pl.ds(wid * EW, EW)])
        pltpu.sync_copy(ssum, out_h.at[wid])

    return pl.kernel(
        body,
        out_type=(_f32(E), _f32(NW, N)),
        mesh=_get_mesh(),
        compiler_params=pltpu.CompilerParams(needs_layout_passes=False),
        scratch_types=[
            pltpu.VMEM((N,), jnp.float32),
            pltpu.VMEM((EW,), jnp.float32),
            pltpu.VMEM((EW,), jnp.int32),
            pltpu.VMEM((EW,), jnp.float32),
            pltpu.VMEM((N,), jnp.float32),
        ],
    )(attn, src2, m)


def _sc_scatter_add(yl, yr, src3):
    rows_per_tile = NP // NS
    zch = 128
    DH = D // 2

    KB = 5

    def body(yl_h, yr_h, src_h, outl_h, outr_h, *scr):
        idxf = scr[0]
        yb = scr[1:1 + KB]
        zb = scr[1 + KB]
        acc = scr[2 + KB]
        sl = scr[3 + KB:3 + 2 * KB]
        cid = lax.axis_index("c")
        sid = lax.axis_index("s")
        wid = cid * NS + sid
        base = wid * EW
        pltpu.sync_copy(src_h.at[wid], idxf)

        def zz(i, c):
            for k2 in range(DH // L):
                zb[i, pl.ds(k2 * L, L)] = jnp.zeros((L,), jnp.float32)
            return c

        lax.fori_loop(0, zch, zz, 0)

        for half in range(2):
            y_h = (yl_h, yr_h)[half]
            out_h = (outl_h, outr_h)[half]
            for k in range(rows_per_tile // zch):
                pltpu.sync_copy(
                    zb, acc.at[pl.ds(sid * rows_per_tile + k * zch, zch)])
            plsc.subcore_barrier()

            def lb(g, carry):
                c0 = g * KB
                lds = [pltpu.async_copy(
                    y_h.at[pl.ds(base + (c0 + j) * CH, CH)], yb[j], sl[j])
                    for j in range(KB)]
                for j in range(KB):
                    lds[j].wait()
                    pltpu.sync_copy(yb[j], acc.at[idxf.at[c0 + j]], add=True)
                return carry

            lax.fori_loop(0, NCH // KB, lb, 0)
            plsc.subcore_barrier()
            for k in range(rows_per_tile // zch):
                r0 = sid * rows_per_tile + k * zch
                pltpu.sync_copy(acc.at[pl.ds(r0, zch)], zb)
                pltpu.sync_copy(zb, out_h.at[pl.ds(cid * NP + r0, zch)])
            if half == 0:
                lax.fori_loop(0, zch, zz, 0)

    return pl.kernel(
        body,
        out_type=(_f32(NC * NP, DH), _f32(NC * NP, DH)),
        mesh=_get_mesh(),
        compiler_params=pltpu.CompilerParams(
            needs_layout_passes=False, use_tc_tiling_on_sc=False),
        scratch_types=[
            pltpu.VMEM((NCH, CH), jnp.int32),
        ] + [pltpu.VMEM((CH, DH), jnp.float32)] * KB + [
            pltpu.VMEM((zch, DH), jnp.float32),
            pltpu.VMEM_SHARED((NP, DH), jnp.float32),
        ] + [pltpu.SemaphoreType.DMA] * KB,
    )(yl, yr, src3)


def _tc_q(atom, wq):
    def body(a_ref, w_ref, o_ref):
        o_ref[...] = jnp.dot(a_ref[...], w_ref[...],
                             preferred_element_type=jnp.float32)

    return pl.pallas_call(body, out_shape=_f32(N, D))(atom, wq)


def _tc_attn(q_src, x_dst, ef, sh, wk1, bk1, wk2, bk2):
    be = 4000
    grid = E // be

    def body(q_ref, x_ref, e_ref, s_ref, w1_ref, b1_ref, w2_ref, b2_ref, o_ref):
        r = jax.nn.relu(jnp.dot(e_ref[...], w1_ref[...],
                                preferred_element_type=jnp.float32) + b1_ref[...])
        g = q_ref[...] * x_ref[...]
        t = lax.dot_general(g, w2_ref[...], (((1,), (1,)), ((), ())),
                            preferred_element_type=jnp.float32,
                            precision=lax.Precision.HIGHEST)
        att = jnp.sum(r * t, axis=-1, keepdims=True)
        att = att + lax.dot_general(g, b2_ref[...], (((1,), (1,)), ((), ())),
                                    preferred_element_type=jnp.float32,
                                    precision=lax.Precision.HIGHEST)
        o_ref[...] = att * s_ref[...]

    return pl.pallas_call(
        body,
        grid=(grid,),
        in_specs=[
            pl.BlockSpec((be, D), lambda i: (i, 0)),
            pl.BlockSpec((be, D), lambda i: (i, 0)),
            pl.BlockSpec((be, H), lambda i: (i, 0)),
            pl.BlockSpec((be, 1), lambda i: (i, 0)),
            pl.BlockSpec((H, H), lambda i: (0, 0)),
            pl.BlockSpec((1, H), lambda i: (0, 0)),
            pl.BlockSpec((H, D), lambda i: (0, 0)),
            pl.BlockSpec((1, D), lambda i: (0, 0)),
        ],
        out_specs=pl.BlockSpec((be, 1), lambda i: (i, 0)),
        out_shape=_f32(E, 1),
    )(q_src, x_dst, ef, sh, wk1, bk1.reshape(1, H), wk2, bk2.reshape(1, D))


def _tc_merge(parts, op):
    def body(p_ref, o_ref):
        if op == "max":
            o_ref[...] = jnp.max(p_ref[...], axis=0, keepdims=True)
        else:
            o_ref[...] = jnp.sum(p_ref[...], axis=0, keepdims=True)

    return pl.pallas_call(body, out_shape=_f32(1, N))(parts)


def _tc_values(ex, sh, x_dst, ef, wv1, bv1, wv2, bv2):
    be = 4000
    grid = E // be

    def body(w_ref, s_ref, x_ref, e_ref, w1_ref, b1_ref, w2_ref, b2_ref,
             ol_ref, or_ref):
        s2 = jax.nn.relu(jnp.dot(e_ref[...], w1_ref[...],
                                 preferred_element_type=jnp.float32) + b1_ref[...])
        vw = jnp.dot(s2, w2_ref[...],
                     preferred_element_type=jnp.float32) + b2_ref[...]
        y = x_ref[...] * vw * (w_ref[...] * s_ref[...])
        ol_ref[...] = y[:, 0:D // 2]
        or_ref[...] = y[:, D // 2:D]

    return pl.pallas_call(
        body,
        grid=(grid,),
        in_specs=[
            pl.BlockSpec((be, 1), lambda i: (i, 0)),
            pl.BlockSpec((be, 1), lambda i: (i, 0)),
            pl.BlockSpec((be, D), lambda i: (i, 0)),
            pl.BlockSpec((be, H), lambda i: (i, 0)),
            pl.BlockSpec((H, H), lambda i: (0, 0)),
            pl.BlockSpec((1, H), lambda i: (0, 0)),
            pl.BlockSpec((H, D), lambda i: (0, 0)),
            pl.BlockSpec((1, D), lambda i: (0, 0)),
        ],
        out_specs=(pl.BlockSpec((be, D // 2), lambda i: (i, 0)),
                   pl.BlockSpec((be, D // 2), lambda i: (i, 0))),
        out_shape=(_f32(E, D // 2), _f32(E, D // 2)),
    )(ex.reshape(E, 1), sh.reshape(E, 1), x_dst, ef, wv1, bv1.reshape(1, H),
      wv2, bv2.reshape(1, D))


def _tc_node_update(atom, updl, updr, ssum_col, gamma, beta, we):
    def body(a_ref, ul_ref, ur_ref, ss_ref, g_ref, b_ref, w_ref, x_ref, sa_ref):
        ul = ul_ref[...]
        ur = ur_ref[...]
        u = jnp.concatenate([ul[0:N] + ul[NP:NP + N],
                             ur[0:N] + ur[NP:NP + N]], axis=1)
        u = u / (ss_ref[...] + 1e-16)
        x = a_ref[...] + u
        mu = jnp.mean(x, axis=0, keepdims=True)
        d = x - mu
        var = jnp.mean(d * d, axis=0, keepdims=True)
        xn = d * lax.rsqrt(var + 1e-5) * g_ref[...] + b_ref[...]
        x_ref[...] = xn
        sa_ref[...] = jnp.dot(xn, w_ref[...], preferred_element_type=jnp.float32)

    return pl.pallas_call(
        body,
        out_shape=(_f32(N, D), _f32(N, H)),
    )(atom, updl, updr, ssum_col, gamma.reshape(1, D), beta.reshape(1, D), we)


def _tc_edge_mlp(ef, sa_dst, sa_src, we1, be1, we2, be2, we3, be3, lg, lb):
    be = 8000
    grid = E // be
    a1 = we1[0:H]
    a2 = we1[H:2 * H]
    a3 = we1[2 * H:3 * H]

    def body(e_ref, d_ref, s_ref, a1_ref, a2_ref, a3_ref, b1_ref,
             w2_ref, b2_ref, w3_ref, b3_ref, g_ref, bb_ref, o_ref):
        efv = e_ref[...]
        h = (jnp.dot(d_ref[...], a1_ref[...], preferred_element_type=jnp.float32)
             + jnp.dot(s_ref[...], a2_ref[...], preferred_element_type=jnp.float32)
             + jnp.dot(efv, a3_ref[...], preferred_element_type=jnp.float32)
             + b1_ref[...])
        h = jax.nn.relu(h)
        h = jax.nn.relu(jnp.dot(h, w2_ref[...],
                                preferred_element_type=jnp.float32) + b2_ref[...])
        h = jnp.dot(h, w3_ref[...], preferred_element_type=jnp.float32) + b3_ref[...]
        e = efv + h
        mu = jnp.mean(e, axis=-1, keepdims=True)
        dv = e - mu
        var = jnp.mean(dv * dv, axis=-1, keepdims=True)
        o_ref[...] = dv * lax.rsqrt(var + 1e-5) * g_ref[...] + bb_ref[...]

    wspec = pl.BlockSpec((H, H), lambda i: (0, 0))
    bspec = pl.BlockSpec((1, H), lambda i: (0, 0))
    espec = pl.BlockSpec((be, H), lambda i: (i, 0))
    return pl.pallas_call(
        body,
        grid=(grid,),
        in_specs=[espec, espec, espec, wspec, wspec, wspec, bspec,
                  wspec, bspec, wspec, bspec, bspec, bspec],
        out_specs=espec,
        out_shape=_f32(E, H),
    )(ef, sa_dst, sa_src, a1, a2, a3, be1.reshape(1, H),
      we2, be2.reshape(1, H), we3, be3.reshape(1, H),
      lg.reshape(1, H), lb.reshape(1, H))


def kernel(atom_features, edge_features, edge_sh, edge_index, W_q, Wk1, bk1,
           Wk2, bk2, Wv1, bv1, Wv2, bv2, bn_gamma, bn_beta, W_e, We1, be1,
           We2, be2, We3, be3, ln_gamma, ln_beta):
    dst = edge_index[0]
    src = edge_index[1]
    src2 = src.reshape(NW, EW)
    src3 = src.reshape(NW, NCH, CH)
    dst3 = dst.reshape(NW, NCH, CH)
    sh_flat = edge_sh.reshape(E)

    qn = _tc_q(atom_features, W_q)
    q_src, x_dst = _sc_gather_pair(qn, atom_features, src3, dst3, D)
    attn = _tc_attn(q_src, x_dst, edge_features, edge_sh, Wk1, bk1, Wk2, bk2)
    attn_flat = attn.reshape(E)
    mpart = _sc_segmax_partials(attn_flat, src2)
    m = _tc_merge(mpart, "max").reshape(N)
    ex, spart = _sc_exp_segsum(attn_flat, src2, m)
    ssum_col = _tc_merge(spart, "sum").reshape(N, 1)
    yl, yr = _tc_values(ex, sh_flat, x_dst, edge_features, Wv1, bv1, Wv2, bv2)
    updl, updr = _sc_scatter_add(yl, yr, src3)
    x_out, sa = _tc_node_update(atom_features, updl, updr, ssum_col,
                                bn_gamma, bn_beta, W_e)
    sa_dst, sa_src = _sc_gather_pair(sa, sa, dst3, src3, H, tc_tiling=False)
    e_out = _tc_edge_mlp(edge_features, sa_dst, sa_src, We1, be1, We2, be2,
                         We3, be3, ln_gamma, ln_beta)
    return (x_out, e_out)


# speedup vs baseline: 4.6312x; 1.4245x over previous
import jax
import jax.numpy as jnp
from jax import lax
from jax.experimental import pallas as pl
from jax.experimental.pallas import tpu as pltpu
from jax.experimental.pallas import tpu_sc as plsc

N = 10000
E = 320000
D = 128
H = 16

NC = 2
NS = 16
NW = NC * NS
EW = E // NW
CH = 40
NCH = EW // CH
NP = 10240
L = 16

def _get_mesh():
    return plsc.VectorSubcoreMesh(core_axis_name="c", subcore_axis_name="s")


def _f32(*shape):
    return jax.ShapeDtypeStruct(shape, jnp.float32)


def _sc_gather_pair(ta, tb, ia3, ib3, dw, tc_tiling=True):
    KB = 5

    def body(ta_h, tb_h, ia_h, ib_h, outa_h, outb_h, *scr):
        idxa, idxb = scr[0], scr[1]
        ba = scr[2:2 + KB]
        bb = scr[2 + KB:2 + 2 * KB]
        sems = scr[2 + 2 * KB:]
        sga = sems[0:KB]
        sgb = sems[KB:2 * KB]
        swa = sems[2 * KB:3 * KB]
        swb = sems[3 * KB:4 * KB]
        cid = lax.axis_index("c")
        sid = lax.axis_index("s")
        wid = cid * NS + sid
        base = wid * EW
        pltpu.sync_copy(ia_h.at[wid], idxa)
        pltpu.sync_copy(ib_h.at[wid], idxb)

        def loop_body(g, carry):
            c0 = g * KB
            gd = []
            for j in range(KB):
                gd.append((
                    pltpu.async_copy(ta_h.at[idxa.at[c0 + j]], ba[j], sga[j]),
                    pltpu.async_copy(tb_h.at[idxb.at[c0 + j]], bb[j], sgb[j]),
                ))
            wd = []
            for j in range(KB):
                gd[j][0].wait()
                gd[j][1].wait()
                wd.append((
                    pltpu.async_copy(
                        ba[j], outa_h.at[pl.ds(base + (c0 + j) * CH, CH)], swa[j]),
                    pltpu.async_copy(
                        bb[j], outb_h.at[pl.ds(base + (c0 + j) * CH, CH)], swb[j]),
                ))
            for j in range(KB):
                wd[j][0].wait()
                wd[j][1].wait()
            return carry

        lax.fori_loop(0, NCH // KB, loop_body, 0)

    return pl.kernel(
        body,
        out_type=(_f32(E, dw), _f32(E, dw)),
        mesh=_get_mesh(),
        compiler_params=pltpu.CompilerParams(
            needs_layout_passes=False, use_tc_tiling_on_sc=tc_tiling),
        scratch_types=[
            pltpu.VMEM((NCH, CH), jnp.int32),
            pltpu.VMEM((NCH, CH), jnp.int32),
        ] + [pltpu.VMEM((CH, dw), jnp.float32)] * (2 * KB)
          + [pltpu.SemaphoreType.DMA] * (4 * KB),
    )(ta, tb, ia3, ib3)


def _sc_segmax_partials(attn, src2):
    def body(attn_h, src_h, out_h, mbuf, abuf, sbuf):
        cid = lax.axis_index("c")
        sid = lax.axis_index("s")
        wid = cid * NS + sid
        pltpu.sync_copy(attn_h.at[pl.ds(wid * EW, EW)], abuf)
        pltpu.sync_copy(src_h.at[wid], sbuf)

        def zero(j, c):
            mbuf[pl.ds(j * L, L)] = jnp.full((L,), -1e30, jnp.float32)
            return c

        lax.fori_loop(0, N // L, zero, 0)
        iot = lax.broadcasted_iota(jnp.int32, (L,), 0)

        def step(j, c):
            idx = sbuf[pl.ds(j * L, L)]
            a = abuf[pl.ds(j * L, L)]
            for l in range(L):
                mk = iot == l
                cur = plsc.load_gather(mbuf, [idx], mask=mk)
                plsc.store_scatter(mbuf, [idx], jnp.maximum(cur, a), mask=mk)
            return c

        lax.fori_loop(0, EW // L, step, 0)
        pltpu.sync_copy(mbuf, out_h.at[wid])

    return pl.kernel(
        body,
        out_type=_f32(NW, N),
        mesh=_get_mesh(),
        compiler_params=pltpu.CompilerParams(needs_layout_passes=False),
        scratch_types=[
            pltpu.VMEM((N,), jnp.float32),
            pltpu.VMEM((EW,), jnp.float32),
            pltpu.VMEM((EW,), jnp.int32),
        ],
    )(attn, src2)


def _sc_exp_segsum(attn, src2, m, sh):
    def body(attn_h, src_h, m_h, sh_h, ex_h, out_h, mfull, abuf, sbuf, hbuf,
             ebuf, ssum):
        cid = lax.axis_index("c")
        sid = lax.axis_index("s")
        wid = cid * NS + sid
        pltpu.sync_copy(m_h, mfull)
        pltpu.sync_copy(attn_h.at[pl.ds(wid * EW, EW)], abuf)
        pltpu.sync_copy(src_h.at[wid], sbuf)
        pltpu.sync_copy(sh_h.at[pl.ds(wid * EW, EW)], hbuf)

        def zero(j, c):
            ssum[pl.ds(j * L, L)] = jnp.zeros((L,), jnp.float32)
            return c

        lax.fori_loop(0, N // L, zero, 0)
        iot = lax.broadcasted_iota(jnp.int32, (L,), 0)

        def step(j, c):
            idx = sbuf[pl.ds(j * L, L)]
            a = abuf[pl.ds(j * L, L)]
            mg = plsc.load_gather(mfull, [idx])
            ex = jnp.exp(a - mg)
            ebuf[pl.ds(j * L, L)] = ex * hbuf[pl.ds(j * L, L)]
            for l in range(L):
                mk = iot == l
                cur = plsc.load_gather(ssum, [idx], mask=mk)
                plsc.store_scatter(ssum, [idx], cur + ex, mask=mk)
            return c

        lax.fori_loop(0, EW // L, step, 0)
        pltpu.sync_copy(ebuf, ex_h.at[pl.ds(wid * EW, EW)])
        pltpu.sync_copy(ssum, out_h.at[wid])

    return pl.kernel(
        body,
        out_type=(_f32(E), _f32(NW, N)),
        mesh=_get_mesh(),
        compiler_params=pltpu.CompilerParams(needs_layout_passes=False),
        scratch_types=[
            pltpu.VMEM((N,), jnp.float32),
            pltpu.VMEM((EW,), jnp.float32),
            pltpu.VMEM((EW,), jnp.int32),
            pltpu.VMEM((EW,), jnp.float32),
            pltpu.VMEM((EW,), jnp.float32),
            pltpu.VMEM((N,), jnp.float32),
        ],
    )(attn, src2, m, sh)


def _sc_scatter_add(y, exsh, src3):
    rows_per_tile = NP // NS
    zch = 128
    DH = D // 2

    KB = 5

    def body(y_h, w_h, src_h, outl_h, outr_h, *scr):
        idxf = scr[0]
        wbuf = scr[1]
        yb = scr[2:2 + KB]
        yh = scr[2 + KB]
        zb = scr[3 + KB]
        acc = scr[4 + KB]
        sl = scr[5 + KB:5 + 2 * KB]
        cid = lax.axis_index("c")
        sid = lax.axis_index("s")
        wid = cid * NS + sid
        base = wid * EW
        pltpu.sync_copy(src_h.at[wid], idxf)
        pltpu.sync_copy(w_h.at[pl.ds(base, EW)], wbuf.at[pl.ds(0, EW)])

        def zz(i, c):
            for k2 in range(DH // L):
                zb[i, pl.ds(k2 * L, L)] = jnp.zeros((L,), jnp.float32)
            return c

        lax.fori_loop(0, zch, zz, 0)

        for half in range(2):
            col0 = half * DH
            out_h = (outl_h, outr_h)[half]
            for k in range(rows_per_tile // zch):
                pltpu.sync_copy(
                    zb, acc.at[pl.ds(sid * rows_per_tile + k * zch, zch)])
            plsc.subcore_barrier()

            def lb(g, carry):
                c0 = g * KB
                lds = [pltpu.async_copy(
                    y_h.at[pl.ds(base + (c0 + j) * CH, CH)], yb[j], sl[j])
                    for j in range(KB)]
                for j in range(KB):
                    lds[j].wait()
                    cbase = (c0 + j) * CH
                    svec = [wbuf[pl.ds(cbase + L * k, L)]
                            for k in range((CH + L - 1) // L)]
                    for r in range(CH):
                        s = svec[r // L][r % L]
                        for k2 in range(DH // L):
                            yh[r, pl.ds(k2 * L, L)] = (
                                yb[j][r, pl.ds(col0 + k2 * L, L)] * s)
                    pltpu.sync_copy(yh, acc.at[idxf.at[c0 + j]], add=True)
                return carry

            lax.fori_loop(0, NCH // KB, lb, 0)
            plsc.subcore_barrier()
            for k in range(rows_per_tile // zch):
                r0 = sid * rows_per_tile + k * zch
                pltpu.sync_copy(acc.at[pl.ds(r0, zch)], zb)
                pltpu.sync_copy(zb, out_h.at[pl.ds(cid * NP + r0, zch)])
            if half == 0:
                lax.fori_loop(0, zch, zz, 0)

    return pl.kernel(
        body,
        out_type=(_f32(NC * NP, DH), _f32(NC * NP, DH)),
        mesh=_get_mesh(),
        compiler_params=pltpu.CompilerParams(
            needs_layout_passes=False, use_tc_tiling_on_sc=False),
        scratch_types=[
            pltpu.VMEM((NCH, CH), jnp.int32),
            pltpu.VMEM((EW + L,), jnp.float32),
        ] + [pltpu.VMEM((CH, D), jnp.float32)] * KB + [
            pltpu.VMEM((CH, DH), jnp.float32),
            pltpu.VMEM((zch, DH), jnp.float32),
            pltpu.VMEM_SHARED((NP, DH), jnp.float32),
        ] + [pltpu.SemaphoreType.DMA] * KB,
    )(y, exsh, src3)


def _tc_q(atom, wq):
    def body(a_ref, w_ref, o_ref):
        o_ref[...] = jnp.dot(a_ref[...], w_ref[...],
                             preferred_element_type=jnp.float32)

    return pl.pallas_call(body, out_shape=_f32(N, D))(atom, wq)


def _tc_attn(q_src, x_dst, ef, sh, wk1, bk1, wk2, bk2):
    be = 4000
    grid = E // be

    def body(q_ref, x_ref, e_ref, s_ref, w1_ref, b1_ref, w2_ref, b2_ref, o_ref):
        r = jax.nn.relu(jnp.dot(e_ref[...], w1_ref[...],
                                preferred_element_type=jnp.float32) + b1_ref[...])
        g = q_ref[...] * x_ref[...]
        t = lax.dot_general(g, w2_ref[...], (((1,), (1,)), ((), ())),
                            preferred_element_type=jnp.float32,
                            precision=lax.Precision.HIGHEST)
        att = jnp.sum(r * t, axis=-1, keepdims=True)
        att = att + lax.dot_general(g, b2_ref[...], (((1,), (1,)), ((), ())),
                                    preferred_element_type=jnp.float32,
                                    precision=lax.Precision.HIGHEST)
        o_ref[...] = att * s_ref[...]

    return pl.pallas_call(
        body,
        grid=(grid,),
        in_specs=[
            pl.BlockSpec((be, D), lambda i: (i, 0)),
            pl.BlockSpec((be, D), lambda i: (i, 0)),
            pl.BlockSpec((be, H), lambda i: (i, 0)),
            pl.BlockSpec((be, 1), lambda i: (i, 0)),
            pl.BlockSpec((H, H), lambda i: (0, 0)),
            pl.BlockSpec((1, H), lambda i: (0, 0)),
            pl.BlockSpec((H, D), lambda i: (0, 0)),
            pl.BlockSpec((1, D), lambda i: (0, 0)),
        ],
        out_specs=pl.BlockSpec((be, 1), lambda i: (i, 0)),
        out_shape=_f32(E, 1),
    )(q_src, x_dst, ef, sh, wk1, bk1.reshape(1, H), wk2, bk2.reshape(1, D))


def _tc_merge(parts, op):
    def body(p_ref, o_ref):
        if op == "max":
            o_ref[...] = jnp.max(p_ref[...], axis=0, keepdims=True)
        else:
            o_ref[...] = jnp.sum(p_ref[...], axis=0, keepdims=True)

    return pl.pallas_call(body, out_shape=_f32(1, N))(parts)


def _tc_values(x_dst, ef, wv1, bv1, wv2, bv2):
    be = 4000
    grid = E // be

    def body(x_ref, e_ref, w1_ref, b1_ref, w2_ref, b2_ref, o_ref):
        s2 = jax.nn.relu(jnp.dot(e_ref[...], w1_ref[...],
                                 preferred_element_type=jnp.float32) + b1_ref[...])
        vw = jnp.dot(s2, w2_ref[...],
                     preferred_element_type=jnp.float32) + b2_ref[...]
        o_ref[...] = x_ref[...] * vw

    return pl.pallas_call(
        body,
        grid=(grid,),
        in_specs=[
            pl.BlockSpec((be, D), lambda i: (i, 0)),
            pl.BlockSpec((be, H), lambda i: (i, 0)),
            pl.BlockSpec((H, H), lambda i: (0, 0)),
            pl.BlockSpec((1, H), lambda i: (0, 0)),
            pl.BlockSpec((H, D), lambda i: (0, 0)),
            pl.BlockSpec((1, D), lambda i: (0, 0)),
        ],
        out_specs=pl.BlockSpec((be, D), lambda i: (i, 0)),
        out_shape=_f32(E, D),
    )(x_dst, ef, wv1, bv1.reshape(1, H), wv2, bv2.reshape(1, D))


def _tc_node_update(atom, updl, updr, ssum_col, gamma, beta, we):
    def body(a_ref, ul_ref, ur_ref, ss_ref, g_ref, b_ref, w_ref, x_ref, sa_ref):
        ul = ul_ref[...]
        ur = ur_ref[...]
        u = jnp.concatenate([ul[0:N] + ul[NP:NP + N],
                             ur[0:N] + ur[NP:NP + N]], axis=1)
        u = u / (ss_ref[...] + 1e-16)
        x = a_ref[...] + u
        mu = jnp.mean(x, axis=0, keepdims=True)
        d = x - mu
        var = jnp.mean(d * d, axis=0, keepdims=True)
        xn = d * lax.rsqrt(var + 1e-5) * g_ref[...] + b_ref[...]
        x_ref[...] = xn
        sa_ref[...] = jnp.dot(xn, w_ref[...], preferred_element_type=jnp.float32)

    return pl.pallas_call(
        body,
        out_shape=(_f32(N, D), _f32(N, H)),
    )(atom, updl, updr, ssum_col, gamma.reshape(1, D), beta.reshape(1, D), we)


def _tc_edge_mlp(ef, sa_dst, sa_src, we1, be1, we2, be2, we3, be3, lg, lb):
    E8 = E // 8
    be = 4000
    grid = E8 // be
    eye8 = jnp.eye(8, dtype=jnp.float32)
    b1 = jnp.kron(eye8, we1[0:H])
    b2 = jnp.kron(eye8, we1[H:2 * H])
    b3 = jnp.kron(eye8, we1[2 * H:3 * H])
    bw2 = jnp.kron(eye8, we2)
    bw3 = jnp.kron(eye8, we3)
    mavg = jnp.kron(eye8, jnp.full((H, H), 1.0 / H, jnp.float32))
    ef8 = ef.reshape(E8, 8 * H)
    sad8 = sa_dst.reshape(E8, 8 * H)
    sas8 = sa_src.reshape(E8, 8 * H)

    def body(e_ref, d_ref, s_ref, b1_ref, b2_ref, b3_ref, bb1_ref,
             w2_ref, bb2_ref, w3_ref, bb3_ref, m_ref, g_ref, bb_ref, o_ref):
        efv = e_ref[...]
        h = (jnp.dot(d_ref[...], b1_ref[...], preferred_element_type=jnp.float32)
             + jnp.dot(s_ref[...], b2_ref[...], preferred_element_type=jnp.float32)
             + jnp.dot(efv, b3_ref[...], preferred_element_type=jnp.float32)
             + bb1_ref[...])
        h = jax.nn.relu(h)
        h = jax.nn.relu(jnp.dot(h, w2_ref[...],
                                preferred_element_type=jnp.float32) + bb2_ref[...])
        h = jnp.dot(h, w3_ref[...], preferred_element_type=jnp.float32) + bb3_ref[...]
        e = efv + h
        mu = jnp.dot(e, m_ref[...], preferred_element_type=jnp.float32,
                     precision=lax.Precision.HIGHEST)
        dv = e - mu
        var = jnp.dot(dv * dv, m_ref[...], preferred_element_type=jnp.float32,
                      precision=lax.Precision.HIGHEST)
        o_ref[...] = dv * lax.rsqrt(var + 1e-5) * g_ref[...] + bb_ref[...]

    wspec = pl.BlockSpec((8 * H, 8 * H), lambda i: (0, 0))
    bspec = pl.BlockSpec((1, 8 * H), lambda i: (0, 0))
    espec = pl.BlockSpec((be, 8 * H), lambda i: (i, 0))
    out8 = pl.pallas_call(
        body,
        grid=(grid,),
        in_specs=[espec, espec, espec, wspec, wspec, wspec, bspec,
                  wspec, bspec, wspec, bspec, wspec, bspec, bspec],
        out_specs=espec,
        out_shape=_f32(E8, 8 * H),
    )(ef8, sad8, sas8, b1, b2, b3, jnp.tile(be1, 8).reshape(1, 8 * H),
      bw2, jnp.tile(be2, 8).reshape(1, 8 * H),
      bw3, jnp.tile(be3, 8).reshape(1, 8 * H),
      mavg, jnp.tile(lg, 8).reshape(1, 8 * H), jnp.tile(lb, 8).reshape(1, 8 * H))
    return out8.reshape(E, H)


def kernel(atom_features, edge_features, edge_sh, edge_index, W_q, Wk1, bk1,
           Wk2, bk2, Wv1, bv1, Wv2, bv2, bn_gamma, bn_beta, W_e, We1, be1,
           We2, be2, We3, be3, ln_gamma, ln_beta):
    dst = edge_index[0]
    src = edge_index[1]
    src2 = src.reshape(NW, EW)
    src3 = src.reshape(NW, NCH, CH)
    dst3 = dst.reshape(NW, NCH, CH)
    sh_flat = edge_sh.reshape(E)

    qn = _tc_q(atom_features, W_q)
    q_src, x_dst = _sc_gather_pair(qn, atom_features, src3, dst3, D)
    attn = _tc_attn(q_src, x_dst, edge_features, edge_sh, Wk1, bk1, Wk2, bk2)
    attn_flat = attn.reshape(E)
    mpart = _sc_segmax_partials(attn_flat, src2)
    m = _tc_merge(mpart, "max").reshape(N)
    exsh, spart = _sc_exp_segsum(attn_flat, src2, m, sh_flat)
    ssum_col = _tc_merge(spart, "sum").reshape(N, 1)
    y = _tc_values(x_dst, edge_features, Wv1, bv1, Wv2, bv2)
    updl, updr = _sc_scatter_add(y, exsh, src3)
    x_out, sa = _tc_node_update(atom_features, updl, updr, ssum_col,
                                bn_gamma, bn_beta, W_e)
    sa_dst, sa_src = _sc_gather_pair(sa, sa, dst3, src3, H, tc_tiling=False)
    e_out = _tc_edge_mlp(edge_features, sa_dst, sa_src, We1, be1, We2, be2,
                         We3, be3, ln_gamma, ln_beta)
    return (x_out, e_out)


# speedup vs baseline: 4.7190x; 1.0189x over previous
import jax
import jax.numpy as jnp
from jax import lax
from jax.experimental import pallas as pl
from jax.experimental.pallas import tpu as pltpu
from jax.experimental.pallas import tpu_sc as plsc

N = 10000
E = 320000
D = 128
H = 16

NC = 2
NS = 16
NW = NC * NS
EW = E // NW
CH = 40
NCH = EW // CH
NP = 10240
L = 16

def _get_mesh():
    return plsc.VectorSubcoreMesh(core_axis_name="c", subcore_axis_name="s")


def _f32(*shape):
    return jax.ShapeDtypeStruct(shape, jnp.float32)


def _sc_gather_pair(ta, tb, ia3, ib3, dw, tc_tiling=True, product=False):
    KB = 5

    def body(ta_h, tb_h, ia_h, ib_h, outa_h, outb_h, *scr):
        idxa, idxb = scr[0], scr[1]
        ba = scr[2:2 + KB]
        bb = scr[2 + KB:2 + 2 * KB]
        sems = scr[2 + 2 * KB:]
        sga = sems[0:KB]
        sgb = sems[KB:2 * KB]
        swa = sems[2 * KB:3 * KB]
        swb = sems[3 * KB:4 * KB]
        cid = lax.axis_index("c")
        sid = lax.axis_index("s")
        wid = cid * NS + sid
        base = wid * EW
        pltpu.sync_copy(ia_h.at[wid], idxa)
        pltpu.sync_copy(ib_h.at[wid], idxb)

        def loop_body(g, carry):
            c0 = g * KB
            gd = []
            for j in range(KB):
                gd.append((
                    pltpu.async_copy(ta_h.at[idxa.at[c0 + j]], ba[j], sga[j]),
                    pltpu.async_copy(tb_h.at[idxb.at[c0 + j]], bb[j], sgb[j]),
                ))
            wd = []
            for j in range(KB):
                gd[j][0].wait()
                gd[j][1].wait()
                if product:
                    def mulrow(r, cc, j=j):
                        for k2 in range(dw // L):
                            sl = pl.ds(k2 * L, L)
                            ba[j][r, sl] = ba[j][r, sl] * bb[j][r, sl]
                        return cc

                    lax.fori_loop(0, CH, mulrow, 0)
                wd.append((
                    pltpu.async_copy(
                        ba[j], outa_h.at[pl.ds(base + (c0 + j) * CH, CH)], swa[j]),
                    pltpu.async_copy(
                        bb[j], outb_h.at[pl.ds(base + (c0 + j) * CH, CH)], swb[j]),
                ))
            for j in range(KB):
                wd[j][0].wait()
                wd[j][1].wait()
            return carry

        lax.fori_loop(0, NCH // KB, loop_body, 0)

    return pl.kernel(
        body,
        out_type=(_f32(E, dw), _f32(E, dw)),
        mesh=_get_mesh(),
        compiler_params=pltpu.CompilerParams(
            needs_layout_passes=False, use_tc_tiling_on_sc=tc_tiling),
        scratch_types=[
            pltpu.VMEM((NCH, CH), jnp.int32),
            pltpu.VMEM((NCH, CH), jnp.int32),
        ] + [pltpu.VMEM((CH, dw), jnp.float32)] * (2 * KB)
          + [pltpu.SemaphoreType.DMA] * (4 * KB),
    )(ta, tb, ia3, ib3)


def _sc_segmax_partials(attn, src2):
    def body(attn_h, src_h, out_h, mbuf, abuf, sbuf):
        cid = lax.axis_index("c")
        sid = lax.axis_index("s")
        wid = cid * NS + sid
        pltpu.sync_copy(attn_h.at[pl.ds(wid * EW, EW)], abuf)
        pltpu.sync_copy(src_h.at[wid], sbuf)

        def zero(j, c):
            mbuf[pl.ds(j * L, L)] = jnp.full((L,), -1e30, jnp.float32)
            return c

        lax.fori_loop(0, N // L, zero, 0)
        iot = lax.broadcasted_iota(jnp.int32, (L,), 0)

        def step(j, c):
            idx = sbuf[pl.ds(j * L, L)]
            a = abuf[pl.ds(j * L, L)]
            for l in range(L):
                mk = iot == l
                cur = plsc.load_gather(mbuf, [idx], mask=mk)
                plsc.store_scatter(mbuf, [idx], jnp.maximum(cur, a), mask=mk)
            return c

        lax.fori_loop(0, EW // L, step, 0)
        pltpu.sync_copy(mbuf, out_h.at[wid])

    return pl.kernel(
        body,
        out_type=_f32(NW, N),
        mesh=_get_mesh(),
        compiler_params=pltpu.CompilerParams(needs_layout_passes=False),
        scratch_types=[
            pltpu.VMEM((N,), jnp.float32),
            pltpu.VMEM((EW,), jnp.float32),
            pltpu.VMEM((EW,), jnp.int32),
        ],
    )(attn, src2)


def _sc_exp_segsum(attn, src2, m, sh):
    def body(attn_h, src_h, m_h, sh_h, ex_h, out_h, mfull, abuf, sbuf, hbuf,
             ebuf, ssum):
        cid = lax.axis_index("c")
        sid = lax.axis_index("s")
        wid = cid * NS + sid
        pltpu.sync_copy(m_h, mfull)
        pltpu.sync_copy(attn_h.at[pl.ds(wid * EW, EW)], abuf)
        pltpu.sync_copy(src_h.at[wid], sbuf)
        pltpu.sync_copy(sh_h.at[pl.ds(wid * EW, EW)], hbuf)

        def zero(j, c):
            ssum[pl.ds(j * L, L)] = jnp.zeros((L,), jnp.float32)
            return c

        lax.fori_loop(0, N // L, zero, 0)
        iot = lax.broadcasted_iota(jnp.int32, (L,), 0)

        def step(j, c):
            idx = sbuf[pl.ds(j * L, L)]
            a = abuf[pl.ds(j * L, L)]
            mg = plsc.load_gather(mfull, [idx])
            ex = jnp.exp(a - mg)
            ebuf[pl.ds(j * L, L)] = ex * hbuf[pl.ds(j * L, L)]
            for l in range(L):
                mk = iot == l
                cur = plsc.load_gather(ssum, [idx], mask=mk)
                plsc.store_scatter(ssum, [idx], cur + ex, mask=mk)
            return c

        lax.fori_loop(0, EW // L, step, 0)
        pltpu.sync_copy(ebuf, ex_h.at[pl.ds(wid * EW, EW)])
        pltpu.sync_copy(ssum, out_h.at[wid])

    return pl.kernel(
        body,
        out_type=(_f32(E), _f32(NW, N)),
        mesh=_get_mesh(),
        compiler_params=pltpu.CompilerParams(needs_layout_passes=False),
        scratch_types=[
            pltpu.VMEM((N,), jnp.float32),
            pltpu.VMEM((EW,), jnp.float32),
            pltpu.VMEM((EW,), jnp.int32),
            pltpu.VMEM((EW,), jnp.float32),
            pltpu.VMEM((EW,), jnp.float32),
            pltpu.VMEM((N,), jnp.float32),
        ],
    )(attn, src2, m, sh)


def _sc_scatter_add(y, exsh, src3):
    rows_per_tile = NP // NS
    zch = 128
    DH = D // 2

    KB = 5

    def body(y_h, w_h, src_h, outl_h, outr_h, *scr):
        idxf = scr[0]
        wbuf = scr[1]
        yb = scr[2:2 + KB]
        yh = scr[2 + KB]
        zb = scr[3 + KB]
        acc = scr[4 + KB]
        sl = scr[5 + KB:5 + 2 * KB]
        cid = lax.axis_index("c")
        sid = lax.axis_index("s")
        wid = cid * NS + sid
        base = wid * EW
        pltpu.sync_copy(src_h.at[wid], idxf)
        pltpu.sync_copy(w_h.at[pl.ds(base, EW)], wbuf.at[pl.ds(0, EW)])

        def zz(i, c):
            for k2 in range(DH // L):
                zb[i, pl.ds(k2 * L, L)] = jnp.zeros((L,), jnp.float32)
            return c

        lax.fori_loop(0, zch, zz, 0)

        for half in range(2):
            col0 = half * DH
            out_h = (outl_h, outr_h)[half]
            for k in range(rows_per_tile // zch):
                pltpu.sync_copy(
                    zb, acc.at[pl.ds(sid * rows_per_tile + k * zch, zch)])
            plsc.subcore_barrier()

            def lb(g, carry):
                c0 = g * KB
                lds = [pltpu.async_copy(
                    y_h.at[pl.ds(base + (c0 + j) * CH, CH)], yb[j], sl[j])
                    for j in range(KB)]
                for j in range(KB):
                    lds[j].wait()
                    cbase = (c0 + j) * CH
                    svec = [wbuf[pl.ds(cbase + L * k, L)]
                            for k in range((CH + L - 1) // L)]
                    for r in range(CH):
                        s = svec[r // L][r % L]
                        for k2 in range(DH // L):
                            yh[r, pl.ds(k2 * L, L)] = (
                                yb[j][r, pl.ds(col0 + k2 * L, L)] * s)
                    pltpu.sync_copy(yh, acc.at[idxf.at[c0 + j]], add=True)
                return carry

            lax.fori_loop(0, NCH // KB, lb, 0)
            plsc.subcore_barrier()
            for k in range(rows_per_tile // zch):
                r0 = sid * rows_per_tile + k * zch
                pltpu.sync_copy(acc.at[pl.ds(r0, zch)], zb)
                pltpu.sync_copy(zb, out_h.at[pl.ds(cid * NP + r0, zch)])
            if half == 0:
                lax.fori_loop(0, zch, zz, 0)

    return pl.kernel(
        body,
        out_type=(_f32(NC * NP, DH), _f32(NC * NP, DH)),
        mesh=_get_mesh(),
        compiler_params=pltpu.CompilerParams(
            needs_layout_passes=False, use_tc_tiling_on_sc=False),
        scratch_types=[
            pltpu.VMEM((NCH, CH), jnp.int32),
            pltpu.VMEM((EW + L,), jnp.float32),
        ] + [pltpu.VMEM((CH, D), jnp.float32)] * KB + [
            pltpu.VMEM((CH, DH), jnp.float32),
            pltpu.VMEM((zch, DH), jnp.float32),
            pltpu.VMEM_SHARED((NP, DH), jnp.float32),
        ] + [pltpu.SemaphoreType.DMA] * KB,
    )(y, exsh, src3)


def _tc_q(atom, wq):
    def body(a_ref, w_ref, o_ref):
        o_ref[...] = jnp.dot(a_ref[...], w_ref[...],
                             preferred_element_type=jnp.float32)

    return pl.pallas_call(body, out_shape=_f32(N, D))(atom, wq)


def _tc_attn(g_arr, ef, sh, wk1, bk1, wk2, bk2):
    be = 4000
    grid = E // be

    def body(q_ref, e_ref, s_ref, w1_ref, b1_ref, w2_ref, b2_ref, o_ref):
        r = jax.nn.relu(jnp.dot(e_ref[...], w1_ref[...],
                                preferred_element_type=jnp.float32) + b1_ref[...])
        g = q_ref[...]
        t = lax.dot_general(g, w2_ref[...], (((1,), (1,)), ((), ())),
                            preferred_element_type=jnp.float32,
                            precision=lax.Precision.HIGHEST)
        att = jnp.sum(r * t, axis=-1, keepdims=True)
        att = att + lax.dot_general(g, b2_ref[...], (((1,), (1,)), ((), ())),
                                    preferred_element_type=jnp.float32,
                                    precision=lax.Precision.HIGHEST)
        o_ref[...] = att * s_ref[...]

    return pl.pallas_call(
        body,
        grid=(grid,),
        in_specs=[
            pl.BlockSpec((be, D), lambda i: (i, 0)),
            pl.BlockSpec((be, H), lambda i: (i, 0)),
            pl.BlockSpec((be, 1), lambda i: (i, 0)),
            pl.BlockSpec((H, H), lambda i: (0, 0)),
            pl.BlockSpec((1, H), lambda i: (0, 0)),
            pl.BlockSpec((H, D), lambda i: (0, 0)),
            pl.BlockSpec((1, D), lambda i: (0, 0)),
        ],
        out_specs=pl.BlockSpec((be, 1), lambda i: (i, 0)),
        out_shape=_f32(E, 1),
    )(g_arr, ef, sh, wk1, bk1.reshape(1, H), wk2, bk2.reshape(1, D))


def _tc_merge(parts, op):
    def body(p_ref, o_ref):
        if op == "max":
            o_ref[...] = jnp.max(p_ref[...], axis=0, keepdims=True)
        else:
            o_ref[...] = jnp.sum(p_ref[...], axis=0, keepdims=True)

    return pl.pallas_call(body, out_shape=_f32(1, N))(parts)


def _tc_values(x_dst, ef, wv1, bv1, wv2, bv2):
    be = 4000
    grid = E // be

    def body(x_ref, e_ref, w1_ref, b1_ref, w2_ref, b2_ref, o_ref):
        s2 = jax.nn.relu(jnp.dot(e_ref[...], w1_ref[...],
                                 preferred_element_type=jnp.float32) + b1_ref[...])
        vw = jnp.dot(s2, w2_ref[...],
                     preferred_element_type=jnp.float32) + b2_ref[...]
        o_ref[...] = x_ref[...] * vw

    return pl.pallas_call(
        body,
        grid=(grid,),
        in_specs=[
            pl.BlockSpec((be, D), lambda i: (i, 0)),
            pl.BlockSpec((be, H), lambda i: (i, 0)),
            pl.BlockSpec((H, H), lambda i: (0, 0)),
            pl.BlockSpec((1, H), lambda i: (0, 0)),
            pl.BlockSpec((H, D), lambda i: (0, 0)),
            pl.BlockSpec((1, D), lambda i: (0, 0)),
        ],
        out_specs=pl.BlockSpec((be, D), lambda i: (i, 0)),
        out_shape=_f32(E, D),
    )(x_dst, ef, wv1, bv1.reshape(1, H), wv2, bv2.reshape(1, D))


def _tc_node_update(atom, updl, updr, ssum_col, gamma, beta, we):
    def body(a_ref, ul_ref, ur_ref, ss_ref, g_ref, b_ref, w_ref, x_ref, sa_ref):
        ul = ul_ref[...]
        ur = ur_ref[...]
        u = jnp.concatenate([ul[0:N] + ul[NP:NP + N],
                             ur[0:N] + ur[NP:NP + N]], axis=1)
        u = u / (ss_ref[...] + 1e-16)
        x = a_ref[...] + u
        mu = jnp.mean(x, axis=0, keepdims=True)
        d = x - mu
        var = jnp.mean(d * d, axis=0, keepdims=True)
        xn = d * lax.rsqrt(var + 1e-5) * g_ref[...] + b_ref[...]
        x_ref[...] = xn
        sa_ref[...] = jnp.dot(xn, w_ref[...], preferred_element_type=jnp.float32)

    return pl.pallas_call(
        body,
        out_shape=(_f32(N, D), _f32(N, H)),
    )(atom, updl, updr, ssum_col, gamma.reshape(1, D), beta.reshape(1, D), we)


def _tc_edge_mlp(ef, sa_dst, sa_src, we1, be1, we2, be2, we3, be3, lg, lb):
    E8 = E // 8
    be = 4000
    grid = E8 // be
    eye8 = jnp.eye(8, dtype=jnp.float32)
    b1 = jnp.kron(eye8, we1[0:H])
    b2 = jnp.kron(eye8, we1[H:2 * H])
    b3 = jnp.kron(eye8, we1[2 * H:3 * H])
    bw2 = jnp.kron(eye8, we2)
    bw3 = jnp.kron(eye8, we3)
    mavg = jnp.kron(eye8, jnp.full((H, H), 1.0 / H, jnp.float32))
    ef8 = ef.reshape(E8, 8 * H)
    sad8 = sa_dst.reshape(E8, 8 * H)
    sas8 = sa_src.reshape(E8, 8 * H)

    def body(e_ref, d_ref, s_ref, b1_ref, b2_ref, b3_ref, bb1_ref,
             w2_ref, bb2_ref, w3_ref, bb3_ref, m_ref, g_ref, bb_ref, o_ref):
        efv = e_ref[...]
        h = (jnp.dot(d_ref[...], b1_ref[...], preferred_element_type=jnp.float32)
             + jnp.dot(s_ref[...], b2_ref[...], preferred_element_type=jnp.float32)
             + jnp.dot(efv, b3_ref[...], preferred_element_type=jnp.float32)
             + bb1_ref[...])
        h = jax.nn.relu(h)
        h = jax.nn.relu(jnp.dot(h, w2_ref[...],
                                preferred_element_type=jnp.float32) + bb2_ref[...])
        h = jnp.dot(h, w3_ref[...], preferred_element_type=jnp.float32) + bb3_ref[...]
        e = efv + h
        mu = jnp.dot(e, m_ref[...], preferred_element_type=jnp.float32,
                     precision=lax.Precision.HIGHEST)
        dv = e - mu
        var = jnp.dot(dv * dv, m_ref[...], preferred_element_type=jnp.float32,
                      precision=lax.Precision.HIGHEST)
        o_ref[...] = dv * lax.rsqrt(var + 1e-5) * g_ref[...] + bb_ref[...]

    wspec = pl.BlockSpec((8 * H, 8 * H), lambda i: (0, 0))
    bspec = pl.BlockSpec((1, 8 * H), lambda i: (0, 0))
    espec = pl.BlockSpec((be, 8 * H), lambda i: (i, 0))
    out8 = pl.pallas_call(
        body,
        grid=(grid,),
        in_specs=[espec, espec, espec, wspec, wspec, wspec, bspec,
                  wspec, bspec, wspec, bspec, wspec, bspec, bspec],
        out_specs=espec,
        out_shape=_f32(E8, 8 * H),
    )(ef8, sad8, sas8, b1, b2, b3, jnp.tile(be1, 8).reshape(1, 8 * H),
      bw2, jnp.tile(be2, 8).reshape(1, 8 * H),
      bw3, jnp.tile(be3, 8).reshape(1, 8 * H),
      mavg, jnp.tile(lg, 8).reshape(1, 8 * H), jnp.tile(lb, 8).reshape(1, 8 * H))
    return out8.reshape(E, H)


def kernel(atom_features, edge_features, edge_sh, edge_index, W_q, Wk1, bk1,
           Wk2, bk2, Wv1, bv1, Wv2, bv2, bn_gamma, bn_beta, W_e, We1, be1,
           We2, be2, We3, be3, ln_gamma, ln_beta):
    dst = edge_index[0]
    src = edge_index[1]
    src2 = src.reshape(NW, EW)
    src3 = src.reshape(NW, NCH, CH)
    dst3 = dst.reshape(NW, NCH, CH)
    sh_flat = edge_sh.reshape(E)

    qn = _tc_q(atom_features, W_q)
    g_arr, x_dst = _sc_gather_pair(qn, atom_features, src3, dst3, D,
                                   product=True)
    attn = _tc_attn(g_arr, edge_features, edge_sh, Wk1, bk1, Wk2, bk2)
    attn_flat = attn.reshape(E)
    mpart = _sc_segmax_partials(attn_flat, src2)
    m = _tc_merge(mpart, "max").reshape(N)
    exsh, spart = _sc_exp_segsum(attn_flat, src2, m, sh_flat)
    ssum_col = _tc_merge(spart, "sum").reshape(N, 1)
    y = _tc_values(x_dst, edge_features, Wv1, bv1, Wv2, bv2)
    updl, updr = _sc_scatter_add(y, exsh, src3)
    x_out, sa = _tc_node_update(atom_features, updl, updr, ssum_col,
                                bn_gamma, bn_beta, W_e)
    sa_dst, sa_src = _sc_gather_pair(sa, sa, dst3, src3, H, tc_tiling=False)
    e_out = _tc_edge_mlp(edge_features, sa_dst, sa_src, We1, be1, We2, be2,
                         We3, be3, ln_gamma, ln_beta)
    return (x_out, e_out)


# speedup vs baseline: 4.8770x; 1.0335x over previous
import jax
import jax.numpy as jnp
from jax import lax
from jax.experimental import pallas as pl
from jax.experimental.pallas import tpu as pltpu
from jax.experimental.pallas import tpu_sc as plsc

N = 10000
E = 320000
D = 128
H = 16

NC = 2
NS = 16
NW = NC * NS
EW = E // NW
CH = 40
NCH = EW // CH
NP = 10240
L = 16

def _get_mesh():
    return plsc.VectorSubcoreMesh(core_axis_name="c", subcore_axis_name="s")


def _f32(*shape):
    return jax.ShapeDtypeStruct(shape, jnp.float32)


def _sc_gather_pair(ta, tb, ia3, ib3, dw, tc_tiling=True, product=False):
    KB = 5

    def body(ta_h, tb_h, ia_h, ib_h, outa_h, outb_h, *scr):
        idxa, idxb = scr[0], scr[1]
        ba = scr[2:2 + KB]
        bb = scr[2 + KB:2 + 2 * KB]
        sems = scr[2 + 2 * KB:]
        sga = sems[0:KB]
        sgb = sems[KB:2 * KB]
        swa = sems[2 * KB:3 * KB]
        swb = sems[3 * KB:4 * KB]
        cid = lax.axis_index("c")
        sid = lax.axis_index("s")
        wid = cid * NS + sid
        base = wid * EW
        pltpu.sync_copy(ia_h.at[wid], idxa)
        pltpu.sync_copy(ib_h.at[wid], idxb)

        def loop_body(g, carry):
            c0 = g * KB
            gd = []
            for j in range(KB):
                gd.append((
                    pltpu.async_copy(ta_h.at[idxa.at[c0 + j]], ba[j], sga[j]),
                    pltpu.async_copy(tb_h.at[idxb.at[c0 + j]], bb[j], sgb[j]),
                ))
            wd = []
            for j in range(KB):
                gd[j][0].wait()
                gd[j][1].wait()
                if product:
                    def mulrow(r, cc, j=j):
                        for k2 in range(dw // L):
                            sl = pl.ds(k2 * L, L)
                            ba[j][r, sl] = ba[j][r, sl] * bb[j][r, sl]
                        return cc

                    lax.fori_loop(0, CH, mulrow, 0)
                wd.append((
                    pltpu.async_copy(
                        ba[j], outa_h.at[pl.ds(base + (c0 + j) * CH, CH)], swa[j]),
                    pltpu.async_copy(
                        bb[j], outb_h.at[pl.ds(base + (c0 + j) * CH, CH)], swb[j]),
                ))
            for j in range(KB):
                wd[j][0].wait()
                wd[j][1].wait()
            return carry

        lax.fori_loop(0, NCH // KB, loop_body, 0)

    return pl.kernel(
        body,
        out_type=(_f32(E, dw), _f32(E, dw)),
        mesh=_get_mesh(),
        compiler_params=pltpu.CompilerParams(
            needs_layout_passes=False, use_tc_tiling_on_sc=tc_tiling),
        scratch_types=[
            pltpu.VMEM((NCH, CH), jnp.int32),
            pltpu.VMEM((NCH, CH), jnp.int32),
        ] + [pltpu.VMEM((CH, dw), jnp.float32)] * (2 * KB)
          + [pltpu.SemaphoreType.DMA] * (4 * KB),
    )(ta, tb, ia3, ib3)


def _sc_segmax_partials(attn, src2):
    def body(attn_h, src_h, out_h, mbuf, abuf, sbuf):
        cid = lax.axis_index("c")
        sid = lax.axis_index("s")
        wid = cid * NS + sid
        pltpu.sync_copy(attn_h.at[pl.ds(wid * EW, EW)], abuf)
        pltpu.sync_copy(src_h.at[wid], sbuf)

        def zero(j, c):
            mbuf[pl.ds(j * L, L)] = jnp.full((L,), -1e30, jnp.float32)
            return c

        lax.fori_loop(0, N // L, zero, 0)
        iot = lax.broadcasted_iota(jnp.int32, (L,), 0)

        def step(j, c):
            idx = sbuf[pl.ds(j * L, L)]
            a = abuf[pl.ds(j * L, L)]
            for l in range(L):
                mk = iot == l
                cur = plsc.load_gather(mbuf, [idx], mask=mk)
                plsc.store_scatter(mbuf, [idx], jnp.maximum(cur, a), mask=mk)
            return c

        lax.fori_loop(0, EW // L, step, 0)
        pltpu.sync_copy(mbuf, out_h.at[wid])

    return pl.kernel(
        body,
        out_type=_f32(NW, N),
        mesh=_get_mesh(),
        compiler_params=pltpu.CompilerParams(needs_layout_passes=False),
        scratch_types=[
            pltpu.VMEM((N,), jnp.float32),
            pltpu.VMEM((EW,), jnp.float32),
            pltpu.VMEM((EW,), jnp.int32),
        ],
    )(attn, src2)


def _sc_exp_segsum(attn, src2, m, sh):
    def body(attn_h, src_h, m_h, sh_h, ex_h, out_h, mfull, abuf, sbuf, hbuf,
             ebuf, ssum):
        cid = lax.axis_index("c")
        sid = lax.axis_index("s")
        wid = cid * NS + sid
        pltpu.sync_copy(m_h, mfull)
        pltpu.sync_copy(attn_h.at[pl.ds(wid * EW, EW)], abuf)
        pltpu.sync_copy(src_h.at[wid], sbuf)
        pltpu.sync_copy(sh_h.at[pl.ds(wid * EW, EW)], hbuf)

        def zero(j, c):
            ssum[pl.ds(j * L, L)] = jnp.zeros((L,), jnp.float32)
            return c

        lax.fori_loop(0, N // L, zero, 0)
        iot = lax.broadcasted_iota(jnp.int32, (L,), 0)

        def step(j, c):
            idx = sbuf[pl.ds(j * L, L)]
            a = abuf[pl.ds(j * L, L)]
            mg = plsc.load_gather(mfull, [idx])
            ex = jnp.exp(a - mg)
            ebuf[pl.ds(j * L, L)] = ex * hbuf[pl.ds(j * L, L)]
            for l in range(L):
                mk = iot == l
                cur = plsc.load_gather(ssum, [idx], mask=mk)
                plsc.store_scatter(ssum, [idx], cur + ex, mask=mk)
            return c

        lax.fori_loop(0, EW // L, step, 0)
        pltpu.sync_copy(ebuf, ex_h.at[pl.ds(wid * EW, EW)])
        pltpu.sync_copy(ssum, out_h.at[wid])

    return pl.kernel(
        body,
        out_type=(_f32(E), _f32(NW, N)),
        mesh=_get_mesh(),
        compiler_params=pltpu.CompilerParams(needs_layout_passes=False),
        scratch_types=[
            pltpu.VMEM((N,), jnp.float32),
            pltpu.VMEM((EW,), jnp.float32),
            pltpu.VMEM((EW,), jnp.int32),
            pltpu.VMEM((EW,), jnp.float32),
            pltpu.VMEM((EW,), jnp.float32),
            pltpu.VMEM((N,), jnp.float32),
        ],
    )(attn, src2, m, sh)


def _sc_scatter_add(y, exsh, src3):
    rows_per_tile = NP // NS
    zch = 128
    DH = D // 2

    KB = 5

    def body(y_h, w_h, src_h, outl_h, outr_h, *scr):
        idxf = scr[0]
        wbuf = scr[1]
        yb = scr[2:2 + KB]
        yh = scr[2 + KB]
        zb = scr[3 + KB]
        acc = scr[4 + KB]
        sl = scr[5 + KB:5 + 2 * KB]
        cid = lax.axis_index("c")
        sid = lax.axis_index("s")
        wid = cid * NS + sid
        base = wid * EW
        pltpu.sync_copy(src_h.at[wid], idxf)
        pltpu.sync_copy(w_h.at[pl.ds(base, EW)], wbuf.at[pl.ds(0, EW)])

        def zz(i, c):
            for k2 in range(DH // L):
                zb[i, pl.ds(k2 * L, L)] = jnp.zeros((L,), jnp.float32)
            return c

        lax.fori_loop(0, zch, zz, 0)

        for half in range(2):
            col0 = half * DH
            out_h = (outl_h, outr_h)[half]
            for k in range(rows_per_tile // zch):
                pltpu.sync_copy(
                    zb, acc.at[pl.ds(sid * rows_per_tile + k * zch, zch)])
            plsc.subcore_barrier()

            def lb(g, carry):
                c0 = g * KB
                lds = [pltpu.async_copy(
                    y_h.at[pl.ds(base + (c0 + j) * CH, CH)], yb[j], sl[j])
                    for j in range(KB)]
                for j in range(KB):
                    lds[j].wait()
                    cbase = (c0 + j) * CH
                    svec = [wbuf[pl.ds(cbase + L * k, L)]
                            for k in range((CH + L - 1) // L)]
                    for r in range(CH):
                        s = svec[r // L][r % L]
                        for k2 in range(DH // L):
                            yh[r, pl.ds(k2 * L, L)] = (
                                yb[j][r, pl.ds(col0 + k2 * L, L)] * s)
                    pltpu.sync_copy(yh, acc.at[idxf.at[c0 + j]], add=True)
                return carry

            lax.fori_loop(0, NCH // KB, lb, 0)
            plsc.subcore_barrier()
            for k in range(rows_per_tile // zch):
                r0 = sid * rows_per_tile + k * zch
                pltpu.sync_copy(acc.at[pl.ds(r0, zch)], zb)
                pltpu.sync_copy(zb, out_h.at[pl.ds(cid * NP + r0, zch)])
            if half == 0:
                lax.fori_loop(0, zch, zz, 0)

    return pl.kernel(
        body,
        out_type=(_f32(NC * NP, DH), _f32(NC * NP, DH)),
        mesh=_get_mesh(),
        compiler_params=pltpu.CompilerParams(
            needs_layout_passes=False, use_tc_tiling_on_sc=False),
        scratch_types=[
            pltpu.VMEM((NCH, CH), jnp.int32),
            pltpu.VMEM((EW + L,), jnp.float32),
        ] + [pltpu.VMEM((CH, D), jnp.float32)] * KB + [
            pltpu.VMEM((CH, DH), jnp.float32),
            pltpu.VMEM((zch, DH), jnp.float32),
            pltpu.VMEM_SHARED((NP, DH), jnp.float32),
        ] + [pltpu.SemaphoreType.DMA] * KB,
    )(y, exsh, src3)


def _tc_q(atom, wq):
    def body(a_ref, w_ref, o_ref):
        o_ref[...] = jnp.dot(a_ref[...], w_ref[...],
                             preferred_element_type=jnp.float32)

    return pl.pallas_call(body, out_shape=_f32(N, D))(atom, wq)


def _tc_attn(g_arr, ef, sh, wk1, bk1, wk2, bk2):
    be = 4000
    grid = E // be

    def body(q_ref, e_ref, s_ref, w1_ref, b1_ref, w2_ref, b2_ref, o_ref):
        r = jax.nn.relu(jnp.dot(e_ref[...], w1_ref[...],
                                preferred_element_type=jnp.float32) + b1_ref[...])
        kw = jnp.dot(r, w2_ref[...],
                     preferred_element_type=jnp.float32) + b2_ref[...]
        att = jnp.sum(q_ref[...] * kw, axis=-1, keepdims=True)
        o_ref[...] = att * s_ref[...]

    return pl.pallas_call(
        body,
        grid=(grid,),
        in_specs=[
            pl.BlockSpec((be, D), lambda i: (i, 0)),
            pl.BlockSpec((be, H), lambda i: (i, 0)),
            pl.BlockSpec((be, 1), lambda i: (i, 0)),
            pl.BlockSpec((H, H), lambda i: (0, 0)),
            pl.BlockSpec((1, H), lambda i: (0, 0)),
            pl.BlockSpec((H, D), lambda i: (0, 0)),
            pl.BlockSpec((1, D), lambda i: (0, 0)),
        ],
        out_specs=pl.BlockSpec((be, 1), lambda i: (i, 0)),
        out_shape=_f32(E, 1),
    )(g_arr, ef, sh, wk1, bk1.reshape(1, H), wk2, bk2.reshape(1, D))


def _tc_merge(parts, op):
    def body(p_ref, o_ref):
        if op == "max":
            o_ref[...] = jnp.max(p_ref[...], axis=0, keepdims=True)
        else:
            o_ref[...] = jnp.sum(p_ref[...], axis=0, keepdims=True)

    return pl.pallas_call(body, out_shape=_f32(1, N))(parts)


def _tc_values(x_dst, ef, wv1, bv1, wv2, bv2):
    be = 4000
    grid = E // be

    def body(x_ref, e_ref, w1_ref, b1_ref, w2_ref, b2_ref, o_ref):
        s2 = jax.nn.relu(jnp.dot(e_ref[...], w1_ref[...],
                                 preferred_element_type=jnp.float32) + b1_ref[...])
        vw = jnp.dot(s2, w2_ref[...],
                     preferred_element_type=jnp.float32) + b2_ref[...]
        o_ref[...] = x_ref[...] * vw

    return pl.pallas_call(
        body,
        grid=(grid,),
        in_specs=[
            pl.BlockSpec((be, D), lambda i: (i, 0)),
            pl.BlockSpec((be, H), lambda i: (i, 0)),
            pl.BlockSpec((H, H), lambda i: (0, 0)),
            pl.BlockSpec((1, H), lambda i: (0, 0)),
            pl.BlockSpec((H, D), lambda i: (0, 0)),
            pl.BlockSpec((1, D), lambda i: (0, 0)),
        ],
        out_specs=pl.BlockSpec((be, D), lambda i: (i, 0)),
        out_shape=_f32(E, D),
    )(x_dst, ef, wv1, bv1.reshape(1, H), wv2, bv2.reshape(1, D))


def _tc_node_update(atom, updl, updr, ssum_col, gamma, beta, we):
    def body(a_ref, ul_ref, ur_ref, ss_ref, g_ref, b_ref, w_ref, x_ref, sa_ref):
        ul = ul_ref[...]
        ur = ur_ref[...]
        u = jnp.concatenate([ul[0:N] + ul[NP:NP + N],
                             ur[0:N] + ur[NP:NP + N]], axis=1)
        u = u / (ss_ref[...] + 1e-16)
        x = a_ref[...] + u
        mu = jnp.mean(x, axis=0, keepdims=True)
        d = x - mu
        var = jnp.mean(d * d, axis=0, keepdims=True)
        xn = d * lax.rsqrt(var + 1e-5) * g_ref[...] + b_ref[...]
        x_ref[...] = xn
        sa_ref[...] = jnp.dot(xn, w_ref[...], preferred_element_type=jnp.float32)

    return pl.pallas_call(
        body,
        out_shape=(_f32(N, D), _f32(N, H)),
    )(atom, updl, updr, ssum_col, gamma.reshape(1, D), beta.reshape(1, D), we)


def _tc_edge_mlp(ef, sa_dst, sa_src, we1, be1, we2, be2, we3, be3, lg, lb):
    E8 = E // 8
    be = 4000
    grid = E8 // be
    eye8 = jnp.eye(8, dtype=jnp.float32)
    b1 = jnp.kron(eye8, we1[0:H])
    b2 = jnp.kron(eye8, we1[H:2 * H])
    b3 = jnp.kron(eye8, we1[2 * H:3 * H])
    bw2 = jnp.kron(eye8, we2)
    bw3 = jnp.kron(eye8, we3)
    mavg = jnp.kron(eye8, jnp.full((H, H), 1.0 / H, jnp.float32))
    ef8 = ef.reshape(E8, 8 * H)
    sad8 = sa_dst.reshape(E8, 8 * H)
    sas8 = sa_src.reshape(E8, 8 * H)

    def body(e_ref, d_ref, s_ref, b1_ref, b2_ref, b3_ref, bb1_ref,
             w2_ref, bb2_ref, w3_ref, bb3_ref, m_ref, g_ref, bb_ref, o_ref):
        efv = e_ref[...]
        h = (jnp.dot(d_ref[...], b1_ref[...], preferred_element_type=jnp.float32)
             + jnp.dot(s_ref[...], b2_ref[...], preferred_element_type=jnp.float32)
             + jnp.dot(efv, b3_ref[...], preferred_element_type=jnp.float32)
             + bb1_ref[...])
        h = jax.nn.relu(h)
        h = jax.nn.relu(jnp.dot(h, w2_ref[...],
                                preferred_element_type=jnp.float32) + bb2_ref[...])
        h = jnp.dot(h, w3_ref[...], preferred_element_type=jnp.float32) + bb3_ref[...]
        e = efv + h
        mu = jnp.dot(e, m_ref[...], preferred_element_type=jnp.float32,
                     precision=lax.Precision.HIGHEST)
        dv = e - mu
        var = jnp.dot(dv * dv, m_ref[...], preferred_element_type=jnp.float32,
                      precision=lax.Precision.HIGHEST)
        o_ref[...] = dv * lax.rsqrt(var + 1e-5) * g_ref[...] + bb_ref[...]

    wspec = pl.BlockSpec((8 * H, 8 * H), lambda i: (0, 0))
    bspec = pl.BlockSpec((1, 8 * H), lambda i: (0, 0))
    espec = pl.BlockSpec((be, 8 * H), lambda i: (i, 0))
    out8 = pl.pallas_call(
        body,
        grid=(grid,),
        in_specs=[espec, espec, espec, wspec, wspec, wspec, bspec,
                  wspec, bspec, wspec, bspec, wspec, bspec, bspec],
        out_specs=espec,
        out_shape=_f32(E8, 8 * H),
    )(ef8, sad8, sas8, b1, b2, b3, jnp.tile(be1, 8).reshape(1, 8 * H),
      bw2, jnp.tile(be2, 8).reshape(1, 8 * H),
      bw3, jnp.tile(be3, 8).reshape(1, 8 * H),
      mavg, jnp.tile(lg, 8).reshape(1, 8 * H), jnp.tile(lb, 8).reshape(1, 8 * H))
    return out8.reshape(E, H)


def kernel(atom_features, edge_features, edge_sh, edge_index, W_q, Wk1, bk1,
           Wk2, bk2, Wv1, bv1, Wv2, bv2, bn_gamma, bn_beta, W_e, We1, be1,
           We2, be2, We3, be3, ln_gamma, ln_beta):
    dst = edge_index[0]
    src = edge_index[1]
    src2 = src.reshape(NW, EW)
    src3 = src.reshape(NW, NCH, CH)
    dst3 = dst.reshape(NW, NCH, CH)
    sh_flat = edge_sh.reshape(E)

    qn = _tc_q(atom_features, W_q)
    g_arr, x_dst = _sc_gather_pair(qn, atom_features, src3, dst3, D,
                                   product=True)
    attn = _tc_attn(g_arr, edge_features, edge_sh, Wk1, bk1, Wk2, bk2)
    attn_flat = attn.reshape(E)
    mpart = _sc_segmax_partials(attn_flat, src2)
    m = _tc_merge(mpart, "max").reshape(N)
    exsh, spart = _sc_exp_segsum(attn_flat, src2, m, sh_flat)
    ssum_col = _tc_merge(spart, "sum").reshape(N, 1)
    y = _tc_values(x_dst, edge_features, Wv1, bv1, Wv2, bv2)
    updl, updr = _sc_scatter_add(y, exsh, src3)
    x_out, sa = _tc_node_update(atom_features, updl, updr, ssum_col,
                                bn_gamma, bn_beta, W_e)
    sa_dst, sa_src = _sc_gather_pair(sa, sa, dst3, src3, H, tc_tiling=False)
    e_out = _tc_edge_mlp(edge_features, sa_dst, sa_src, We1, be1, We2, be2,
                         We3, be3, ln_gamma, ln_beta)
    return (x_out, e_out)


# speedup vs baseline: 5.0351x; 1.0324x over previous
import jax
import jax.numpy as jnp
from jax import lax
from jax.experimental import pallas as pl
from jax.experimental.pallas import tpu as pltpu
from jax.experimental.pallas import tpu_sc as plsc

N = 10000
E = 320000
D = 128
H = 16

NC = 2
NS = 16
NW = NC * NS
EW = E // NW
CH = 40
NCH = EW // CH
NP = 10240
L = 16

def _get_mesh():
    return plsc.VectorSubcoreMesh(core_axis_name="c", subcore_axis_name="s")


def _f32(*shape):
    return jax.ShapeDtypeStruct(shape, jnp.float32)


def _sc_gather_pair(ta, tb, ia3, ib3, dw, tc_tiling=True, product=False):
    KB = 5

    def body(ta_h, tb_h, ia_h, ib_h, outa_h, outb_h, *scr):
        idxa, idxb = scr[0], scr[1]
        ba = scr[2:2 + KB]
        bb = scr[2 + KB:2 + 2 * KB]
        sems = scr[2 + 2 * KB:]
        sga = sems[0:KB]
        sgb = sems[KB:2 * KB]
        swa = sems[2 * KB:3 * KB]
        swb = sems[3 * KB:4 * KB]
        cid = lax.axis_index("c")
        sid = lax.axis_index("s")
        wid = cid * NS + sid
        base = wid * EW
        pltpu.sync_copy(ia_h.at[wid], idxa)
        pltpu.sync_copy(ib_h.at[wid], idxb)

        def loop_body(g, carry):
            c0 = g * KB
            gd = []
            for j in range(KB):
                gd.append((
                    pltpu.async_copy(ta_h.at[idxa.at[c0 + j]], ba[j], sga[j]),
                    pltpu.async_copy(tb_h.at[idxb.at[c0 + j]], bb[j], sgb[j]),
                ))
            wd = []
            for j in range(KB):
                gd[j][0].wait()
                gd[j][1].wait()
                if product:
                    def mulrow(r, cc, j=j):
                        for k2 in range(dw // L):
                            sl = pl.ds(k2 * L, L)
                            ba[j][r, sl] = ba[j][r, sl] * bb[j][r, sl]
                        return cc

                    lax.fori_loop(0, CH, mulrow, 0)
                wd.append((
                    pltpu.async_copy(
                        ba[j], outa_h.at[pl.ds(base + (c0 + j) * CH, CH)], swa[j]),
                    pltpu.async_copy(
                        bb[j], outb_h.at[pl.ds(base + (c0 + j) * CH, CH)], swb[j]),
                ))
            for j in range(KB):
                wd[j][0].wait()
                wd[j][1].wait()
            return carry

        lax.fori_loop(0, NCH // KB, loop_body, 0)

    return pl.kernel(
        body,
        out_type=(_f32(E, dw), _f32(E, dw)),
        mesh=_get_mesh(),
        compiler_params=pltpu.CompilerParams(
            needs_layout_passes=False, use_tc_tiling_on_sc=tc_tiling),
        scratch_types=[
            pltpu.VMEM((NCH, CH), jnp.int32),
            pltpu.VMEM((NCH, CH), jnp.int32),
        ] + [pltpu.VMEM((CH, dw), jnp.float32)] * (2 * KB)
          + [pltpu.SemaphoreType.DMA] * (4 * KB),
    )(ta, tb, ia3, ib3)


def _sc_segmax_partials(attn, src2):
    def body(attn_h, src_h, out_h, mbuf, abuf, sbuf):
        cid = lax.axis_index("c")
        sid = lax.axis_index("s")
        wid = cid * NS + sid
        pltpu.sync_copy(attn_h.at[pl.ds(wid * EW, EW)], abuf)
        pltpu.sync_copy(src_h.at[wid], sbuf)

        def zero(j, c):
            mbuf[pl.ds(j * L, L)] = jnp.full((L,), -1e30, jnp.float32)
            return c

        lax.fori_loop(0, N // L, zero, 0)
        iot = lax.broadcasted_iota(jnp.int32, (L,), 0)

        def step(j, c):
            idx = sbuf[pl.ds(j * L, L)]
            a = abuf[pl.ds(j * L, L)]
            for l in range(L):
                mk = iot == l
                cur = plsc.load_gather(mbuf, [idx], mask=mk)
                plsc.store_scatter(mbuf, [idx], jnp.maximum(cur, a), mask=mk)
            return c

        lax.fori_loop(0, EW // L, step, 0)
        pltpu.sync_copy(mbuf, out_h.at[wid])

    return pl.kernel(
        body,
        out_type=_f32(NW, N),
        mesh=_get_mesh(),
        compiler_params=pltpu.CompilerParams(needs_layout_passes=False),
        scratch_types=[
            pltpu.VMEM((N,), jnp.float32),
            pltpu.VMEM((EW,), jnp.float32),
            pltpu.VMEM((EW,), jnp.int32),
        ],
    )(attn, src2)


def _sc_exp_segsum(attn, src2, m, sh):
    def body(attn_h, src_h, m_h, sh_h, ex_h, out_h, mfull, abuf, sbuf, hbuf,
             ebuf, ssum):
        cid = lax.axis_index("c")
        sid = lax.axis_index("s")
        wid = cid * NS + sid
        pltpu.sync_copy(m_h, mfull)
        pltpu.sync_copy(attn_h.at[pl.ds(wid * EW, EW)], abuf)
        pltpu.sync_copy(src_h.at[wid], sbuf)
        pltpu.sync_copy(sh_h.at[pl.ds(wid * EW, EW)], hbuf)

        def zero(j, c):
            ssum[pl.ds(j * L, L)] = jnp.zeros((L,), jnp.float32)
            return c

        lax.fori_loop(0, N // L, zero, 0)
        iot = lax.broadcasted_iota(jnp.int32, (L,), 0)

        def step(j, c):
            idx = sbuf[pl.ds(j * L, L)]
            a = abuf[pl.ds(j * L, L)]
            mg = plsc.load_gather(mfull, [idx])
            ex = jnp.exp(a - mg)
            ebuf[pl.ds(j * L, L)] = ex * hbuf[pl.ds(j * L, L)]
            for l in range(L):
                mk = iot == l
                cur = plsc.load_gather(ssum, [idx], mask=mk)
                plsc.store_scatter(ssum, [idx], cur + ex, mask=mk)
            return c

        lax.fori_loop(0, EW // L, step, 0)
        pltpu.sync_copy(ebuf, ex_h.at[pl.ds(wid * EW, EW)])
        pltpu.sync_copy(ssum, out_h.at[wid])

    return pl.kernel(
        body,
        out_type=(_f32(E), _f32(NW, N)),
        mesh=_get_mesh(),
        compiler_params=pltpu.CompilerParams(needs_layout_passes=False),
        scratch_types=[
            pltpu.VMEM((N,), jnp.float32),
            pltpu.VMEM((EW,), jnp.float32),
            pltpu.VMEM((EW,), jnp.int32),
            pltpu.VMEM((EW,), jnp.float32),
            pltpu.VMEM((EW,), jnp.float32),
            pltpu.VMEM((N,), jnp.float32),
        ],
    )(attn, src2, m, sh)


def _sc_scatter_add(y, exsh, src3):
    rows_per_tile = NP // NS
    zch = 128
    DH = D // 2

    KB = 5

    def body(y_h, w_h, src_h, outl_h, outr_h, *scr):
        idxf = scr[0]
        wbuf = scr[1]
        yb = scr[2:2 + KB]
        yh = scr[2 + KB]
        zb = scr[3 + KB]
        acc = scr[4 + KB]
        sl = scr[5 + KB:5 + 2 * KB]
        cid = lax.axis_index("c")
        sid = lax.axis_index("s")
        wid = cid * NS + sid
        base = wid * EW
        pltpu.sync_copy(src_h.at[wid], idxf)
        pltpu.sync_copy(w_h.at[pl.ds(base, EW)], wbuf.at[pl.ds(0, EW)])

        def zz(i, c):
            for k2 in range(DH // L):
                zb[i, pl.ds(k2 * L, L)] = jnp.zeros((L,), jnp.float32)
            return c

        lax.fori_loop(0, zch, zz, 0)

        for half in range(2):
            col0 = half * DH
            out_h = (outl_h, outr_h)[half]
            for k in range(rows_per_tile // zch):
                pltpu.sync_copy(
                    zb, acc.at[pl.ds(sid * rows_per_tile + k * zch, zch)])
            plsc.subcore_barrier()

            def lb(g, carry):
                c0 = g * KB
                lds = [pltpu.async_copy(
                    y_h.at[pl.ds(base + (c0 + j) * CH, CH), pl.ds(col0, DH)],
                    yb[j], sl[j])
                    for j in range(KB)]
                for j in range(KB):
                    lds[j].wait()
                    cbase = (c0 + j) * CH
                    svec = [wbuf[pl.ds(cbase + L * k, L)]
                            for k in range((CH + L - 1) // L)]
                    for r in range(CH):
                        s = svec[r // L][r % L]
                        for k2 in range(DH // L):
                            yh[r, pl.ds(k2 * L, L)] = (
                                yb[j][r, pl.ds(k2 * L, L)] * s)
                    pltpu.sync_copy(yh, acc.at[idxf.at[c0 + j]], add=True)
                return carry

            lax.fori_loop(0, NCH // KB, lb, 0)
            plsc.subcore_barrier()
            for k in range(rows_per_tile // zch):
                r0 = sid * rows_per_tile + k * zch
                pltpu.sync_copy(acc.at[pl.ds(r0, zch)], zb)
                pltpu.sync_copy(zb, out_h.at[pl.ds(cid * NP + r0, zch)])
            if half == 0:
                lax.fori_loop(0, zch, zz, 0)

    return pl.kernel(
        body,
        out_type=(_f32(NC * NP, DH), _f32(NC * NP, DH)),
        mesh=_get_mesh(),
        compiler_params=pltpu.CompilerParams(
            needs_layout_passes=False, use_tc_tiling_on_sc=False),
        scratch_types=[
            pltpu.VMEM((NCH, CH), jnp.int32),
            pltpu.VMEM((EW + L,), jnp.float32),
        ] + [pltpu.VMEM((CH, DH), jnp.float32)] * KB + [
            pltpu.VMEM((CH, DH), jnp.float32),
            pltpu.VMEM((zch, DH), jnp.float32),
            pltpu.VMEM_SHARED((NP, DH), jnp.float32),
        ] + [pltpu.SemaphoreType.DMA] * KB,
    )(y, exsh, src3)


def _tc_q(atom, wq):
    def body(a_ref, w_ref, o_ref):
        o_ref[...] = jnp.dot(a_ref[...], w_ref[...],
                             preferred_element_type=jnp.float32)

    return pl.pallas_call(body, out_shape=_f32(N, D))(atom, wq)


def _tc_attn(g_arr, ef, sh, wk1, bk1, wk2, bk2):
    be = 4000
    grid = E // be

    def body(q_ref, e_ref, s_ref, w1_ref, b1_ref, w2_ref, b2_ref, o_ref):
        r = jax.nn.relu(jnp.dot(e_ref[...], w1_ref[...],
                                preferred_element_type=jnp.float32) + b1_ref[...])
        kw = jnp.dot(r, w2_ref[...],
                     preferred_element_type=jnp.float32) + b2_ref[...]
        att = jnp.sum(q_ref[...] * kw, axis=-1, keepdims=True)
        o_ref[...] = att * s_ref[...]

    return pl.pallas_call(
        body,
        grid=(grid,),
        in_specs=[
            pl.BlockSpec((be, D), lambda i: (i, 0)),
            pl.BlockSpec((be, H), lambda i: (i, 0)),
            pl.BlockSpec((be, 1), lambda i: (i, 0)),
            pl.BlockSpec((H, H), lambda i: (0, 0)),
            pl.BlockSpec((1, H), lambda i: (0, 0)),
            pl.BlockSpec((H, D), lambda i: (0, 0)),
            pl.BlockSpec((1, D), lambda i: (0, 0)),
        ],
        out_specs=pl.BlockSpec((be, 1), lambda i: (i, 0)),
        out_shape=_f32(E, 1),
    )(g_arr, ef, sh, wk1, bk1.reshape(1, H), wk2, bk2.reshape(1, D))


def _tc_merge(parts, op):
    def body(p_ref, o_ref):
        if op == "max":
            o_ref[...] = jnp.max(p_ref[...], axis=0, keepdims=True)
        else:
            o_ref[...] = jnp.sum(p_ref[...], axis=0, keepdims=True)

    return pl.pallas_call(body, out_shape=_f32(1, N))(parts)


def _tc_values(x_dst, ef, wv1, bv1, wv2, bv2):
    be = 4000
    grid = E // be

    def body(x_ref, e_ref, w1_ref, b1_ref, w2_ref, b2_ref, o_ref):
        s2 = jax.nn.relu(jnp.dot(e_ref[...], w1_ref[...],
                                 preferred_element_type=jnp.float32) + b1_ref[...])
        vw = jnp.dot(s2, w2_ref[...],
                     preferred_element_type=jnp.float32) + b2_ref[...]
        o_ref[...] = x_ref[...] * vw

    return pl.pallas_call(
        body,
        grid=(grid,),
        in_specs=[
            pl.BlockSpec((be, D), lambda i: (i, 0)),
            pl.BlockSpec((be, H), lambda i: (i, 0)),
            pl.BlockSpec((H, H), lambda i: (0, 0)),
            pl.BlockSpec((1, H), lambda i: (0, 0)),
            pl.BlockSpec((H, D), lambda i: (0, 0)),
            pl.BlockSpec((1, D), lambda i: (0, 0)),
        ],
        out_specs=pl.BlockSpec((be, D), lambda i: (i, 0)),
        out_shape=_f32(E, D),
    )(x_dst, ef, wv1, bv1.reshape(1, H), wv2, bv2.reshape(1, D))


def _tc_node_update(atom, updl, updr, ssum_col, gamma, beta, we):
    def body(a_ref, ul_ref, ur_ref, ss_ref, g_ref, b_ref, w_ref, x_ref, sa_ref):
        ul = ul_ref[...]
        ur = ur_ref[...]
        u = jnp.concatenate([ul[0:N] + ul[NP:NP + N],
                             ur[0:N] + ur[NP:NP + N]], axis=1)
        u = u / (ss_ref[...] + 1e-16)
        x = a_ref[...] + u
        mu = jnp.mean(x, axis=0, keepdims=True)
        d = x - mu
        var = jnp.mean(d * d, axis=0, keepdims=True)
        xn = d * lax.rsqrt(var + 1e-5) * g_ref[...] + b_ref[...]
        x_ref[...] = xn
        sa_ref[...] = jnp.dot(xn, w_ref[...], preferred_element_type=jnp.float32)

    return pl.pallas_call(
        body,
        out_shape=(_f32(N, D), _f32(N, H)),
    )(atom, updl, updr, ssum_col, gamma.reshape(1, D), beta.reshape(1, D), we)


def _tc_edge_mlp(ef, sa_dst, sa_src, we1, be1, we2, be2, we3, be3, lg, lb):
    E8 = E // 8
    be = 4000
    grid = E8 // be
    eye8 = jnp.eye(8, dtype=jnp.float32)
    b1 = jnp.kron(eye8, we1[0:H])
    b2 = jnp.kron(eye8, we1[H:2 * H])
    b3 = jnp.kron(eye8, we1[2 * H:3 * H])
    bw2 = jnp.kron(eye8, we2)
    bw3 = jnp.kron(eye8, we3)
    mavg = jnp.kron(eye8, jnp.full((H, H), 1.0 / H, jnp.float32))
    ef8 = ef.reshape(E8, 8 * H)
    sad8 = sa_dst.reshape(E8, 8 * H)
    sas8 = sa_src.reshape(E8, 8 * H)

    def body(e_ref, d_ref, s_ref, b1_ref, b2_ref, b3_ref, bb1_ref,
             w2_ref, bb2_ref, w3_ref, bb3_ref, m_ref, g_ref, bb_ref, o_ref):
        efv = e_ref[...]
        h = (jnp.dot(d_ref[...], b1_ref[...], preferred_element_type=jnp.float32)
             + jnp.dot(s_ref[...], b2_ref[...], preferred_element_type=jnp.float32)
             + jnp.dot(efv, b3_ref[...], preferred_element_type=jnp.float32)
             + bb1_ref[...])
        h = jax.nn.relu(h)
        h = jax.nn.relu(jnp.dot(h, w2_ref[...],
                                preferred_element_type=jnp.float32) + bb2_ref[...])
        h = jnp.dot(h, w3_ref[...], preferred_element_type=jnp.float32) + bb3_ref[...]
        e = efv + h
        mu = jnp.dot(e, m_ref[...], preferred_element_type=jnp.float32,
                     precision=lax.Precision.HIGHEST)
        dv = e - mu
        var = jnp.dot(dv * dv, m_ref[...], preferred_element_type=jnp.float32,
                      precision=lax.Precision.HIGHEST)
        o_ref[...] = dv * lax.rsqrt(var + 1e-5) * g_ref[...] + bb_ref[...]

    wspec = pl.BlockSpec((8 * H, 8 * H), lambda i: (0, 0))
    bspec = pl.BlockSpec((1, 8 * H), lambda i: (0, 0))
    espec = pl.BlockSpec((be, 8 * H), lambda i: (i, 0))
    out8 = pl.pallas_call(
        body,
        grid=(grid,),
        in_specs=[espec, espec, espec, wspec, wspec, wspec, bspec,
                  wspec, bspec, wspec, bspec, wspec, bspec, bspec],
        out_specs=espec,
        out_shape=_f32(E8, 8 * H),
    )(ef8, sad8, sas8, b1, b2, b3, jnp.tile(be1, 8).reshape(1, 8 * H),
      bw2, jnp.tile(be2, 8).reshape(1, 8 * H),
      bw3, jnp.tile(be3, 8).reshape(1, 8 * H),
      mavg, jnp.tile(lg, 8).reshape(1, 8 * H), jnp.tile(lb, 8).reshape(1, 8 * H))
    return out8.reshape(E, H)


def kernel(atom_features, edge_features, edge_sh, edge_index, W_q, Wk1, bk1,
           Wk2, bk2, Wv1, bv1, Wv2, bv2, bn_gamma, bn_beta, W_e, We1, be1,
           We2, be2, We3, be3, ln_gamma, ln_beta):
    dst = edge_index[0]
    src = edge_index[1]
    src2 = src.reshape(NW, EW)
    src3 = src.reshape(NW, NCH, CH)
    dst3 = dst.reshape(NW, NCH, CH)
    sh_flat = edge_sh.reshape(E)

    qn = _tc_q(atom_features, W_q)
    g_arr, x_dst = _sc_gather_pair(qn, atom_features, src3, dst3, D,
                                   product=True)
    attn = _tc_attn(g_arr, edge_features, edge_sh, Wk1, bk1, Wk2, bk2)
    attn_flat = attn.reshape(E)
    mpart = _sc_segmax_partials(attn_flat, src2)
    m = _tc_merge(mpart, "max").reshape(N)
    exsh, spart = _sc_exp_segsum(attn_flat, src2, m, sh_flat)
    ssum_col = _tc_merge(spart, "sum").reshape(N, 1)
    y = _tc_values(x_dst, edge_features, Wv1, bv1, Wv2, bv2)
    updl, updr = _sc_scatter_add(y, exsh, src3)
    x_out, sa = _tc_node_update(atom_features, updl, updr, ssum_col,
                                bn_gamma, bn_beta, W_e)
    sa_dst, sa_src = _sc_gather_pair(sa, sa, dst3, src3, H, tc_tiling=False)
    e_out = _tc_edge_mlp(edge_features, sa_dst, sa_src, We1, be1, We2, be2,
                         We3, be3, ln_gamma, ln_beta)
    return (x_out, e_out)
